# Initial kernel scaffold; baseline (speedup 1.0000x reference)
import jax
import jax.numpy as jnp
from jax.experimental import pallas as pl


def kernel(z, edge_attr, mlp_W1, mlp_b1, mlp_W2, mlp_b2, conv1_W, conv1_b, conv2_W, conv2_b, edge_index):
    raise NotImplementedError("write your pallas kernel here")


# speedup vs baseline: 17.4900x; 17.4900x over previous
import functools

import jax
import jax.numpy as jnp
from jax import lax
from jax.experimental import pallas as pl
from jax.experimental.pallas import tpu as pltpu
from jax.experimental.pallas import tpu_sc as plsc

N = 10000
F = 16
HID = 64
OUTW = 32
E = 320000
MLPH = 256

NC, NS = 2, 16
NW = NC * NS
EPW = E // NW
CHUNK = 128
NFULL = EPW // CHUNK
TAIL = EPW - NFULL * CHUNK
PT = 640
NP = NS * PT
DUMMY = N

MLP_BLK = 6400
MLP_STEPS = (N * F) // MLP_BLK

_mesh = plsc.VectorSubcoreMesh(
    core_axis_name="c", subcore_axis_name="s", num_cores=NC, num_subcores=NS
)


def _fill_f32(ref, n, val):
    v = jnp.full((16,), val, jnp.float32)
    for i in range(n // 16):
        ref[pl.ds(16 * i, 16)] = v


@functools.partial(
    pl.kernel,
    out_type=jax.ShapeDtypeStruct((NC, NP), jnp.float32),
    mesh=_mesh,
    scratch_types=[
        pltpu.VMEM((1, CHUNK), jnp.int32),
        pltpu.VMEM((CHUNK,), jnp.float32),
        pltpu.VMEM((PT,), jnp.float32),
        pltpu.VMEM_SHARED((NP,), jnp.float32),
    ],
    compiler_params=pltpu.CompilerParams(use_tc_tiling_on_sc=False),
)
def _hist(col_hbm, out_hbm, cidx, ones_v, zeros_v, acc):
    c = lax.axis_index("c")
    s = lax.axis_index("s")
    wid = c * NS + s
    _fill_f32(ones_v, CHUNK, 1.0)
    _fill_f32(zeros_v, PT, 0.0)
    pltpu.sync_copy(zeros_v, acc.at[pl.ds(s * PT, PT)])
    plsc.subcore_barrier()
    base = wid * EPW

    def body(i, carry):
        pltpu.sync_copy(col_hbm.at[pl.ds(base + i * CHUNK, CHUNK)], cidx.at[0])
        pltpu.sync_copy(ones_v, acc.at[cidx.at[0]], add=True)
        return carry

    lax.fori_loop(0, NFULL, body, 0)
    pltpu.sync_copy(
        col_hbm.at[pl.ds(base + NFULL * CHUNK, TAIL)], cidx.at[0, pl.ds(0, TAIL)]
    )
    pad = jnp.full((16,), DUMMY, jnp.int32)
    for i in range(TAIL // 16, CHUNK // 16):
        cidx[0, pl.ds(16 * i, 16)] = pad
    pltpu.sync_copy(ones_v, acc.at[cidx.at[0]], add=True)
    plsc.subcore_barrier()
    pltpu.sync_copy(acc.at[pl.ds(s * PT, PT)], out_hbm.at[c, pl.ds(s * PT, PT)])


def _make_agg(w):
    @functools.partial(
        pl.kernel,
        out_type=jax.ShapeDtypeStruct((NC, NP, w), jnp.float32),
        mesh=_mesh,
        scratch_types=[
            pltpu.VMEM((1, CHUNK), jnp.int32),
            pltpu.VMEM((1, CHUNK), jnp.int32),
            pltpu.VMEM((CHUNK, w), jnp.float32),
            pltpu.VMEM_SHARED((NP, w), jnp.float32),
            pltpu.SemaphoreType.DMA,
        ],
        compiler_params=pltpu.CompilerParams(use_tc_tiling_on_sc=False),
    )
    def agg(y_hbm, row_hbm, col_hbm, out_hbm, ridx, cidx, rows, acc, sem):
        c = lax.axis_index("c")
        s = lax.axis_index("s")
        wid = c * NS + s
        zv = jnp.zeros((16,), jnp.float32)
        for r in range(CHUNK):
            for j in range(w // 16):
                rows[r, pl.ds(16 * j, 16)] = zv
        for k in range(PT // CHUNK):
            pltpu.sync_copy(rows, acc.at[pl.ds(s * PT + k * CHUNK, CHUNK)])
        plsc.subcore_barrier()
        base = wid * EPW

        def body(i, carry):
            off = base + i * CHUNK
            pltpu.sync_copy(row_hbm.at[pl.ds(off, CHUNK)], ridx.at[0])
            pltpu.sync_copy(col_hbm.at[pl.ds(off, CHUNK)], cidx.at[0])
            pltpu.async_copy(y_hbm.at[ridx.at[0]], rows, sem).wait()
            pltpu.sync_copy(rows, acc.at[cidx.at[0]], add=True)
            return carry

        lax.fori_loop(0, NFULL, body, 0)
        off = base + NFULL * CHUNK
        pltpu.sync_copy(row_hbm.at[pl.ds(off, TAIL)], ridx.at[0, pl.ds(0, TAIL)])
        pltpu.sync_copy(col_hbm.at[pl.ds(off, TAIL)], cidx.at[0, pl.ds(0, TAIL)])
        zpad = jnp.zeros((16,), jnp.int32)
        dpad = jnp.full((16,), DUMMY, jnp.int32)
        for i in range(TAIL // 16, CHUNK // 16):
            ridx[0, pl.ds(16 * i, 16)] = zpad
            cidx[0, pl.ds(16 * i, 16)] = dpad
        pltpu.async_copy(y_hbm.at[ridx.at[0]], rows, sem).wait()
        pltpu.sync_copy(rows, acc.at[cidx.at[0]], add=True)
        plsc.subcore_barrier()
        pltpu.sync_copy(
            acc.at[pl.ds(s * PT, PT)], out_hbm.at[c, pl.ds(s * PT, PT)]
        )

    return agg


_agg16 = _make_agg(F)
_agg32 = _make_agg(OUTW)


def _mlp_body(z_ref, w1_ref, b1_ref, w2_ref, b2_ref, o_ref):
    h1 = jnp.dot(z_ref[...], w1_ref[...], preferred_element_type=jnp.float32)
    h1 = jnp.maximum(h1 + b1_ref[...], 0.0)
    h2 = jnp.dot(h1, w2_ref[...], preferred_element_type=jnp.float32)
    o_ref[...] = jnp.maximum(h2 + b2_ref[...], 0.0)


def _mlp(z, w1, b1, w2, b2):
    return pl.pallas_call(
        _mlp_body,
        grid=(MLP_STEPS,),
        in_specs=[
            pl.BlockSpec((1, F), lambda i: (0, 0)),
            pl.BlockSpec((F, MLPH), lambda i: (0, 0)),
            pl.BlockSpec((1, MLPH), lambda i: (0, 0)),
            pl.BlockSpec((MLPH, MLP_BLK), lambda i: (0, i)),
            pl.BlockSpec((1, MLP_BLK), lambda i: (0, i)),
        ],
        out_specs=pl.BlockSpec((1, MLP_BLK), lambda i: (0, i)),
        out_shape=jax.ShapeDtypeStruct((1, N * F), jnp.float32),
    )(z, w1, b1, w2, b2)


def _scale_body(d0_ref, d1_ref, x_ref, y_ref, dis_ref):
    deg = d0_ref[...] + d1_ref[...] + 1.0
    dis = lax.rsqrt(deg)
    dis10 = lax.slice(dis, (0, 0), (N, 1))
    dis_ref[...] = dis10
    y_ref[...] = dis10 * x_ref[...]


def _scale(d0, d1, x2d):
    return pl.pallas_call(
        _scale_body,
        grid=(1,),
        in_specs=[
            pl.BlockSpec((NP, 1), lambda i: (0, 0)),
            pl.BlockSpec((NP, 1), lambda i: (0, 0)),
            pl.BlockSpec((N, F), lambda i: (0, 0)),
        ],
        out_specs=[
            pl.BlockSpec((N, F), lambda i: (0, 0)),
            pl.BlockSpec((N, 1), lambda i: (0, 0)),
        ],
        out_shape=[
            jax.ShapeDtypeStruct((N, F), jnp.float32),
            jax.ShapeDtypeStruct((N, 1), jnp.float32),
        ],
    )(d0, d1, x2d)


def _conv_body(a0_ref, a1_ref, y_ref, dis_ref, w1c_ref, b1c_ref, w2c_ref, y2_ref):
    dis = dis_ref[...]
    s1 = dis * (a0_ref[...] + a1_ref[...] + y_ref[...])
    out1 = jnp.dot(s1, w1c_ref[...], preferred_element_type=jnp.float32)
    out1 = jnp.maximum(out1 + b1c_ref[...], 0.0)
    y2 = jnp.dot(out1, w2c_ref[...], preferred_element_type=jnp.float32)
    y2_ref[...] = dis * y2


def _conv(a0, a1, y, dis, w1c, b1c, w2c):
    return pl.pallas_call(
        _conv_body,
        grid=(1,),
        in_specs=[
            pl.BlockSpec((N, F), lambda i: (0, 0)),
            pl.BlockSpec((N, F), lambda i: (0, 0)),
            pl.BlockSpec((N, F), lambda i: (0, 0)),
            pl.BlockSpec((N, 1), lambda i: (0, 0)),
            pl.BlockSpec((F, HID), lambda i: (0, 0)),
            pl.BlockSpec((1, HID), lambda i: (0, 0)),
            pl.BlockSpec((HID, OUTW), lambda i: (0, 0)),
        ],
        out_specs=pl.BlockSpec((N, OUTW), lambda i: (0, 0)),
        out_shape=jax.ShapeDtypeStruct((N, OUTW), jnp.float32),
    )(a0, a1, y, dis, w1c, b1c, w2c)


def _final_body(b0_ref, b1_ref, y2_ref, dis_ref, b2c_ref, o_ref):
    o_ref[...] = (
        dis_ref[...] * (b0_ref[...] + b1_ref[...] + y2_ref[...]) + b2c_ref[...]
    )


def _final(b0, b1, y2, dis, b2c):
    return pl.pallas_call(
        _final_body,
        grid=(1,),
        in_specs=[
            pl.BlockSpec((N, OUTW), lambda i: (0, 0)),
            pl.BlockSpec((N, OUTW), lambda i: (0, 0)),
            pl.BlockSpec((N, OUTW), lambda i: (0, 0)),
            pl.BlockSpec((N, 1), lambda i: (0, 0)),
            pl.BlockSpec((1, OUTW), lambda i: (0, 0)),
        ],
        out_specs=pl.BlockSpec((N, OUTW), lambda i: (0, 0)),
        out_shape=jax.ShapeDtypeStruct((N, OUTW), jnp.float32),
    )(b0, b1, y2, dis, b2c)


def kernel(z, edge_attr, mlp_W1, mlp_b1, mlp_W2, mlp_b2,
           conv1_W, conv1_b, conv2_W, conv2_b, edge_index):
    del edge_attr
    row = edge_index[0]
    col = edge_index[1]

    degp = _hist(col)
    d0 = degp[0].reshape(NP, 1)
    d1 = degp[1].reshape(NP, 1)

    x_flat = _mlp(z, mlp_W1, mlp_b1.reshape(1, MLPH), mlp_W2,
                  mlp_b2.reshape(1, N * F))
    x2d = x_flat.reshape(N, F)

    y, dis = _scale(d0, d1, x2d)

    aggp = _agg16(y, row, col)
    y2 = _conv(aggp[0, :N], aggp[1, :N], y, dis,
               conv1_W, conv1_b.reshape(1, HID), conv2_W)

    agg2p = _agg32(y2, row, col)
    out = _final(agg2p[0, :N], agg2p[1, :N], y2, dis,
                 conv2_b.reshape(1, OUTW))
    return out


# speedup vs baseline: 24.8917x; 1.4232x over previous
import functools

import jax
import jax.numpy as jnp
from jax import lax
from jax.experimental import pallas as pl
from jax.experimental.pallas import tpu as pltpu
from jax.experimental.pallas import tpu_sc as plsc

N = 10000
F = 16
HID = 64
OUTW = 32
E = 320000
MLPH = 256

NC, NS = 2, 16
NW = NC * NS
EPW = E // NW
CHUNK = 128
NFULL = EPW // CHUNK
TAIL = EPW - NFULL * CHUNK
PT = 640
NP = NS * PT
DUMMY = N
NCH = NFULL + 1

MLP_BLK = 6400
MLP_STEPS = (N * F) // MLP_BLK

_mesh = plsc.VectorSubcoreMesh(
    core_axis_name="c", subcore_axis_name="s", num_cores=NC, num_subcores=NS
)


def _fill_f32(ref, n, val):
    v = jnp.full((16,), val, jnp.float32)
    for i in range(n // 16):
        ref[pl.ds(16 * i, 16)] = v


@functools.partial(
    pl.kernel,
    out_type=jax.ShapeDtypeStruct((NC, NP), jnp.float32),
    mesh=_mesh,
    scratch_types=[
        pltpu.VMEM((NCH, CHUNK), jnp.int32),
        pltpu.VMEM((CHUNK,), jnp.float32),
        pltpu.VMEM((PT,), jnp.float32),
        pltpu.VMEM_SHARED((NP,), jnp.float32),
        pltpu.SemaphoreType.DMA,
    ],
    compiler_params=pltpu.CompilerParams(use_tc_tiling_on_sc=False),
)
def _hist(col_hbm, out_hbm, cidx_all, ones_v, zeros_v, acc, sem):
    c = lax.axis_index("c")
    s = lax.axis_index("s")
    wid = c * NS + s
    base = wid * EPW
    descs = []
    for k in range(NFULL):
        descs.append(
            pltpu.async_copy(
                col_hbm.at[pl.ds(base + k * CHUNK, CHUNK)], cidx_all.at[k], sem
            )
        )
    descs.append(
        pltpu.async_copy(
            col_hbm.at[pl.ds(base + NFULL * CHUNK, TAIL)],
            cidx_all.at[NFULL, pl.ds(0, TAIL)],
            sem,
        )
    )
    _fill_f32(ones_v, CHUNK, 1.0)
    _fill_f32(zeros_v, PT, 0.0)
    pltpu.sync_copy(zeros_v, acc.at[pl.ds(s * PT, PT)])
    for d in descs:
        d.wait()
    pad = jnp.full((16,), DUMMY, jnp.int32)
    for i in range(TAIL // 16, CHUNK // 16):
        cidx_all[NFULL, pl.ds(16 * i, 16)] = pad
    plsc.subcore_barrier()

    def body(i, carry):
        pltpu.sync_copy(ones_v, acc.at[cidx_all.at[i]], add=True)
        return carry

    lax.fori_loop(0, NCH, body, 0)
    plsc.subcore_barrier()
    pltpu.sync_copy(acc.at[pl.ds(s * PT, PT)], out_hbm.at[c, pl.ds(s * PT, PT)])


def _make_agg(w):
    @functools.partial(
        pl.kernel,
        out_type=jax.ShapeDtypeStruct((NC, NP, w), jnp.float32),
        mesh=_mesh,
        scratch_types=[
            pltpu.VMEM((NCH, CHUNK), jnp.int32),
            pltpu.VMEM((NCH, CHUNK), jnp.int32),
            pltpu.VMEM((2, CHUNK, w), jnp.float32),
            pltpu.VMEM_SHARED((NP, w), jnp.float32),
            pltpu.SemaphoreType.DMA,
            pltpu.SemaphoreType.DMA,
        ],
        compiler_params=pltpu.CompilerParams(use_tc_tiling_on_sc=False),
    )
    def agg(y_hbm, row_hbm, col_hbm, out_hbm, ridx_all, cidx_all, rows, acc,
            semi, semg):
        c = lax.axis_index("c")
        s = lax.axis_index("s")
        wid = c * NS + s
        base = wid * EPW
        descs = []
        for k in range(NFULL):
            off = base + k * CHUNK
            descs.append(
                pltpu.async_copy(row_hbm.at[pl.ds(off, CHUNK)], ridx_all.at[k], semi)
            )
            descs.append(
                pltpu.async_copy(col_hbm.at[pl.ds(off, CHUNK)], cidx_all.at[k], semi)
            )
        off = base + NFULL * CHUNK
        descs.append(
            pltpu.async_copy(
                row_hbm.at[pl.ds(off, TAIL)], ridx_all.at[NFULL, pl.ds(0, TAIL)], semi
            )
        )
        descs.append(
            pltpu.async_copy(
                col_hbm.at[pl.ds(off, TAIL)], cidx_all.at[NFULL, pl.ds(0, TAIL)], semi
            )
        )
        zv = jnp.zeros((16,), jnp.float32)
        for r in range(CHUNK):
            for j in range(w // 16):
                rows[0, r, pl.ds(16 * j, 16)] = zv
        for k in range(PT // CHUNK):
            pltpu.sync_copy(
                rows.at[0], acc.at[pl.ds(s * PT + k * CHUNK, CHUNK)]
            )
        for d in descs:
            d.wait()
        zpad = jnp.zeros((16,), jnp.int32)
        dpad = jnp.full((16,), DUMMY, jnp.int32)
        for i in range(TAIL // 16, CHUNK // 16):
            ridx_all[NFULL, pl.ds(16 * i, 16)] = zpad
            cidx_all[NFULL, pl.ds(16 * i, 16)] = dpad
        plsc.subcore_barrier()

        pltpu.async_copy(y_hbm.at[ridx_all.at[0]], rows.at[0], semg)

        def body(i, carry):
            p = lax.rem(i, 2)
            pltpu.make_async_copy(
                y_hbm.at[ridx_all.at[i]], rows.at[p], semg
            ).wait()

            @pl.when(i + 1 < NCH)
            def _():
                pltpu.async_copy(
                    y_hbm.at[ridx_all.at[i + 1]], rows.at[1 - p], semg
                )

            pltpu.sync_copy(rows.at[p], acc.at[cidx_all.at[i]], add=True)
            return carry

        lax.fori_loop(0, NCH, body, 0)
        plsc.subcore_barrier()
        pltpu.sync_copy(
            acc.at[pl.ds(s * PT, PT)], out_hbm.at[c, pl.ds(s * PT, PT)]
        )

    return agg


_agg16 = _make_agg(F)
_agg32 = _make_agg(OUTW)


def _mlp_body(z_ref, w1_ref, b1_ref, w2_ref, b2_ref, o_ref):
    h1 = jnp.dot(z_ref[...], w1_ref[...], preferred_element_type=jnp.float32)
    h1 = jnp.maximum(h1 + b1_ref[...], 0.0)
    h2 = jnp.dot(h1, w2_ref[...], preferred_element_type=jnp.float32)
    o_ref[...] = jnp.maximum(h2 + b2_ref[...], 0.0)


def _mlp(z, w1, b1, w2, b2):
    return pl.pallas_call(
        _mlp_body,
        grid=(MLP_STEPS,),
        in_specs=[
            pl.BlockSpec((1, F), lambda i: (0, 0)),
            pl.BlockSpec((F, MLPH), lambda i: (0, 0)),
            pl.BlockSpec((1, MLPH), lambda i: (0, 0)),
            pl.BlockSpec((MLPH, MLP_BLK), lambda i: (0, i)),
            pl.BlockSpec((1, MLP_BLK), lambda i: (0, i)),
        ],
        out_specs=pl.BlockSpec((1, MLP_BLK), lambda i: (0, i)),
        out_shape=jax.ShapeDtypeStruct((1, N * F), jnp.float32),
    )(z, w1, b1, w2, b2)


def _scale_body(d0_ref, d1_ref, x_ref, y_ref, dis_ref):
    deg = d0_ref[...] + d1_ref[...] + 1.0
    dis = lax.rsqrt(deg)
    dis10 = lax.slice(dis, (0, 0), (N, 1))
    dis_ref[...] = dis10
    y_ref[...] = dis10 * x_ref[...]


def _scale(d0, d1, x2d):
    return pl.pallas_call(
        _scale_body,
        grid=(1,),
        in_specs=[
            pl.BlockSpec((NP, 1), lambda i: (0, 0)),
            pl.BlockSpec((NP, 1), lambda i: (0, 0)),
            pl.BlockSpec((N, F), lambda i: (0, 0)),
        ],
        out_specs=[
            pl.BlockSpec((N, F), lambda i: (0, 0)),
            pl.BlockSpec((N, 1), lambda i: (0, 0)),
        ],
        out_shape=[
            jax.ShapeDtypeStruct((N, F), jnp.float32),
            jax.ShapeDtypeStruct((N, 1), jnp.float32),
        ],
    )(d0, d1, x2d)


def _conv_body(a0_ref, a1_ref, y_ref, dis_ref, w1c_ref, b1c_ref, w2c_ref, y2_ref):
    dis = dis_ref[...]
    s1 = dis * (a0_ref[...] + a1_ref[...] + y_ref[...])
    out1 = jnp.dot(s1, w1c_ref[...], preferred_element_type=jnp.float32)
    out1 = jnp.maximum(out1 + b1c_ref[...], 0.0)
    y2 = jnp.dot(out1, w2c_ref[...], preferred_element_type=jnp.float32)
    y2_ref[...] = dis * y2


def _conv(a0, a1, y, dis, w1c, b1c, w2c):
    return pl.pallas_call(
        _conv_body,
        grid=(1,),
        in_specs=[
            pl.BlockSpec((N, F), lambda i: (0, 0)),
            pl.BlockSpec((N, F), lambda i: (0, 0)),
            pl.BlockSpec((N, F), lambda i: (0, 0)),
            pl.BlockSpec((N, 1), lambda i: (0, 0)),
            pl.BlockSpec((F, HID), lambda i: (0, 0)),
            pl.BlockSpec((1, HID), lambda i: (0, 0)),
            pl.BlockSpec((HID, OUTW), lambda i: (0, 0)),
        ],
        out_specs=pl.BlockSpec((N, OUTW), lambda i: (0, 0)),
        out_shape=jax.ShapeDtypeStruct((N, OUTW), jnp.float32),
    )(a0, a1, y, dis, w1c, b1c, w2c)


def _final_body(b0_ref, b1_ref, y2_ref, dis_ref, b2c_ref, o_ref):
    o_ref[...] = (
        dis_ref[...] * (b0_ref[...] + b1_ref[...] + y2_ref[...]) + b2c_ref[...]
    )


def _final(b0, b1, y2, dis, b2c):
    return pl.pallas_call(
        _final_body,
        grid=(1,),
        in_specs=[
            pl.BlockSpec((N, OUTW), lambda i: (0, 0)),
            pl.BlockSpec((N, OUTW), lambda i: (0, 0)),
            pl.BlockSpec((N, OUTW), lambda i: (0, 0)),
            pl.BlockSpec((N, 1), lambda i: (0, 0)),
            pl.BlockSpec((1, OUTW), lambda i: (0, 0)),
        ],
        out_specs=pl.BlockSpec((N, OUTW), lambda i: (0, 0)),
        out_shape=jax.ShapeDtypeStruct((N, OUTW), jnp.float32),
    )(b0, b1, y2, dis, b2c)


def kernel(z, edge_attr, mlp_W1, mlp_b1, mlp_W2, mlp_b2,
           conv1_W, conv1_b, conv2_W, conv2_b, edge_index):
    del edge_attr
    row = edge_index[0]
    col = edge_index[1]

    degp = _hist(col)
    d0 = degp[0].reshape(NP, 1)
    d1 = degp[1].reshape(NP, 1)

    x_flat = _mlp(z, mlp_W1, mlp_b1.reshape(1, MLPH), mlp_W2,
                  mlp_b2.reshape(1, N * F))
    x2d = x_flat.reshape(N, F)

    y, dis = _scale(d0, d1, x2d)

    aggp = _agg16(y, row, col)
    y2 = _conv(aggp[0, :N], aggp[1, :N], y, dis,
               conv1_W, conv1_b.reshape(1, HID), conv2_W)

    agg2p = _agg32(y2, row, col)
    out = _final(agg2p[0, :N], agg2p[1, :N], y2, dis,
                 conv2_b.reshape(1, OUTW))
    return out


# speedup vs baseline: 29.9885x; 1.2048x over previous
import functools

import jax
import jax.numpy as jnp
from jax import lax
from jax.experimental import pallas as pl
from jax.experimental.pallas import tpu as pltpu
from jax.experimental.pallas import tpu_sc as plsc

N = 10000
F = 16
HID = 64
OUTW = 32
E = 320000
MLPH = 256

NC, NS = 2, 16
NW = NC * NS
EPW = E // NW
CHUNK = 128
NFULL = EPW // CHUNK
TAIL = EPW - NFULL * CHUNK
PT = 640
NP = NS * PT
DUMMY = N
NCH = NFULL + 1

MLP_BLK = 6400
MLP_STEPS = (N * F) // MLP_BLK

_mesh = plsc.VectorSubcoreMesh(
    core_axis_name="c", subcore_axis_name="s", num_cores=NC, num_subcores=NS
)


def _fill_f32(ref, n, val):
    v = jnp.full((16,), val, jnp.float32)
    for i in range(n // 16):
        ref[pl.ds(16 * i, 16)] = v


@functools.partial(
    pl.kernel,
    out_type=jax.ShapeDtypeStruct((NC, NP), jnp.float32),
    mesh=_mesh,
    scratch_types=[
        pltpu.VMEM((NCH, CHUNK), jnp.int32),
        pltpu.VMEM((CHUNK,), jnp.float32),
        pltpu.VMEM((PT,), jnp.float32),
        pltpu.VMEM_SHARED((NP,), jnp.float32),
        pltpu.SemaphoreType.DMA,
    ],
    compiler_params=pltpu.CompilerParams(use_tc_tiling_on_sc=False),
)
def _hist(col_hbm, out_hbm, cidx_all, ones_v, zeros_v, acc, sem):
    c = lax.axis_index("c")
    s = lax.axis_index("s")
    wid = c * NS + s
    base = wid * EPW
    descs = []
    for k in range(NFULL):
        descs.append(
            pltpu.async_copy(
                col_hbm.at[pl.ds(base + k * CHUNK, CHUNK)], cidx_all.at[k], sem
            )
        )
    descs.append(
        pltpu.async_copy(
            col_hbm.at[pl.ds(base + NFULL * CHUNK, TAIL)],
            cidx_all.at[NFULL, pl.ds(0, TAIL)],
            sem,
        )
    )
    _fill_f32(ones_v, CHUNK, 1.0)
    _fill_f32(zeros_v, PT, 0.0)
    pltpu.sync_copy(zeros_v, acc.at[pl.ds(s * PT, PT)])
    for d in descs:
        d.wait()
    pad = jnp.full((16,), DUMMY, jnp.int32)
    for i in range(TAIL // 16, CHUNK // 16):
        cidx_all[NFULL, pl.ds(16 * i, 16)] = pad
    plsc.subcore_barrier()

    def body(i, carry):
        pltpu.sync_copy(ones_v, acc.at[cidx_all.at[i]], add=True)
        return carry

    lax.fori_loop(0, NCH, body, 0)
    plsc.subcore_barrier()
    pltpu.sync_copy(acc.at[pl.ds(s * PT, PT)], out_hbm.at[c, pl.ds(s * PT, PT)])


def _make_agg(w):
    @functools.partial(
        pl.kernel,
        out_type=jax.ShapeDtypeStruct((NC, NP, w), jnp.float32),
        mesh=_mesh,
        scratch_types=[
            pltpu.VMEM((NCH, CHUNK), jnp.int32),
            pltpu.VMEM((NCH, CHUNK), jnp.int32),
            pltpu.VMEM((4, CHUNK, w), jnp.float32),
            pltpu.VMEM_SHARED((NP, w), jnp.float32),
            pltpu.SemaphoreType.DMA,
            pltpu.SemaphoreType.DMA,
        ],
        compiler_params=pltpu.CompilerParams(use_tc_tiling_on_sc=False),
    )
    def agg(y_hbm, row_hbm, col_hbm, out_hbm, ridx_all, cidx_all, rows, acc,
            semi, semg):
        c = lax.axis_index("c")
        s = lax.axis_index("s")
        wid = c * NS + s
        base = wid * EPW
        descs = []
        for k in range(NFULL):
            off = base + k * CHUNK
            descs.append(
                pltpu.async_copy(row_hbm.at[pl.ds(off, CHUNK)], ridx_all.at[k], semi)
            )
            descs.append(
                pltpu.async_copy(col_hbm.at[pl.ds(off, CHUNK)], cidx_all.at[k], semi)
            )
        off = base + NFULL * CHUNK
        descs.append(
            pltpu.async_copy(
                row_hbm.at[pl.ds(off, TAIL)], ridx_all.at[NFULL, pl.ds(0, TAIL)], semi
            )
        )
        descs.append(
            pltpu.async_copy(
                col_hbm.at[pl.ds(off, TAIL)], cidx_all.at[NFULL, pl.ds(0, TAIL)], semi
            )
        )
        zv = jnp.zeros((16,), jnp.float32)
        for r in range(CHUNK):
            for j in range(w // 16):
                rows[0, r, pl.ds(16 * j, 16)] = zv
        for k in range(PT // CHUNK):
            pltpu.sync_copy(
                rows.at[0], acc.at[pl.ds(s * PT + k * CHUNK, CHUNK)]
            )
        for d in descs:
            d.wait()
        zpad = jnp.zeros((16,), jnp.int32)
        dpad = jnp.full((16,), DUMMY, jnp.int32)
        for i in range(TAIL // 16, CHUNK // 16):
            ridx_all[NFULL, pl.ds(16 * i, 16)] = zpad
            cidx_all[NFULL, pl.ds(16 * i, 16)] = dpad
        plsc.subcore_barrier()

        for k in range(3):
            pltpu.async_copy(y_hbm.at[ridx_all.at[k]], rows.at[k], semg)

        def body(i, carry):
            p = lax.rem(i, 4)
            pltpu.make_async_copy(
                y_hbm.at[ridx_all.at[i]], rows.at[p], semg
            ).wait()

            @pl.when(i + 3 < NCH)
            def _():
                pltpu.async_copy(
                    y_hbm.at[ridx_all.at[i + 3]], rows.at[lax.rem(i + 3, 4)], semg
                )

            pltpu.sync_copy(rows.at[p], acc.at[cidx_all.at[i]], add=True)
            return carry

        lax.fori_loop(0, NCH, body, 0)
        plsc.subcore_barrier()
        pltpu.sync_copy(
            acc.at[pl.ds(s * PT, PT)], out_hbm.at[c, pl.ds(s * PT, PT)]
        )

    return agg


_agg16 = _make_agg(F)
_agg32 = _make_agg(OUTW)


def _mlp_body(z_ref, w1_ref, b1_ref, w2_ref, b2_ref, o_ref):
    h1 = jnp.dot(z_ref[...], w1_ref[...], preferred_element_type=jnp.float32)
    h1 = jnp.maximum(h1 + b1_ref[...], 0.0)
    h2 = jnp.dot(h1, w2_ref[...], preferred_element_type=jnp.float32)
    o_ref[...] = jnp.maximum(h2 + b2_ref[...], 0.0)


def _mlp(z, w1, b1, w2, b2):
    return pl.pallas_call(
        _mlp_body,
        grid=(MLP_STEPS,),
        in_specs=[
            pl.BlockSpec((1, F), lambda i: (0, 0)),
            pl.BlockSpec((F, MLPH), lambda i: (0, 0)),
            pl.BlockSpec((1, MLPH), lambda i: (0, 0)),
            pl.BlockSpec((MLPH, MLP_BLK), lambda i: (0, i)),
            pl.BlockSpec((1, MLP_BLK), lambda i: (0, i)),
        ],
        out_specs=pl.BlockSpec((1, MLP_BLK), lambda i: (0, i)),
        out_shape=jax.ShapeDtypeStruct((1, N * F), jnp.float32),
    )(z, w1, b1, w2, b2)


def _scale_body(d0_ref, d1_ref, x_ref, y_ref, dis_ref):
    deg = d0_ref[...] + d1_ref[...] + 1.0
    dis = lax.rsqrt(deg)
    dis10 = lax.slice(dis, (0, 0), (N, 1))
    dis_ref[...] = dis10
    y_ref[...] = dis10 * x_ref[...]


def _scale(d0, d1, x2d):
    return pl.pallas_call(
        _scale_body,
        grid=(1,),
        in_specs=[
            pl.BlockSpec((NP, 1), lambda i: (0, 0)),
            pl.BlockSpec((NP, 1), lambda i: (0, 0)),
            pl.BlockSpec((N, F), lambda i: (0, 0)),
        ],
        out_specs=[
            pl.BlockSpec((N, F), lambda i: (0, 0)),
            pl.BlockSpec((N, 1), lambda i: (0, 0)),
        ],
        out_shape=[
            jax.ShapeDtypeStruct((N, F), jnp.float32),
            jax.ShapeDtypeStruct((N, 1), jnp.float32),
        ],
    )(d0, d1, x2d)


def _conv_body(a0_ref, a1_ref, y_ref, dis_ref, w1c_ref, b1c_ref, w2c_ref, y2_ref):
    dis = dis_ref[...]
    s1 = dis * (a0_ref[...] + a1_ref[...] + y_ref[...])
    out1 = jnp.dot(s1, w1c_ref[...], preferred_element_type=jnp.float32)
    out1 = jnp.maximum(out1 + b1c_ref[...], 0.0)
    y2 = jnp.dot(out1, w2c_ref[...], preferred_element_type=jnp.float32)
    y2_ref[...] = dis * y2


def _conv(a0, a1, y, dis, w1c, b1c, w2c):
    return pl.pallas_call(
        _conv_body,
        grid=(1,),
        in_specs=[
            pl.BlockSpec((N, F), lambda i: (0, 0)),
            pl.BlockSpec((N, F), lambda i: (0, 0)),
            pl.BlockSpec((N, F), lambda i: (0, 0)),
            pl.BlockSpec((N, 1), lambda i: (0, 0)),
            pl.BlockSpec((F, HID), lambda i: (0, 0)),
            pl.BlockSpec((1, HID), lambda i: (0, 0)),
            pl.BlockSpec((HID, OUTW), lambda i: (0, 0)),
        ],
        out_specs=pl.BlockSpec((N, OUTW), lambda i: (0, 0)),
        out_shape=jax.ShapeDtypeStruct((N, OUTW), jnp.float32),
    )(a0, a1, y, dis, w1c, b1c, w2c)


def _final_body(b0_ref, b1_ref, y2_ref, dis_ref, b2c_ref, o_ref):
    o_ref[...] = (
        dis_ref[...] * (b0_ref[...] + b1_ref[...] + y2_ref[...]) + b2c_ref[...]
    )


def _final(b0, b1, y2, dis, b2c):
    return pl.pallas_call(
        _final_body,
        grid=(1,),
        in_specs=[
            pl.BlockSpec((N, OUTW), lambda i: (0, 0)),
            pl.BlockSpec((N, OUTW), lambda i: (0, 0)),
            pl.BlockSpec((N, OUTW), lambda i: (0, 0)),
            pl.BlockSpec((N, 1), lambda i: (0, 0)),
            pl.BlockSpec((1, OUTW), lambda i: (0, 0)),
        ],
        out_specs=pl.BlockSpec((N, OUTW), lambda i: (0, 0)),
        out_shape=jax.ShapeDtypeStruct((N, OUTW), jnp.float32),
    )(b0, b1, y2, dis, b2c)


def kernel(z, edge_attr, mlp_W1, mlp_b1, mlp_W2, mlp_b2,
           conv1_W, conv1_b, conv2_W, conv2_b, edge_index):
    del edge_attr
    row = edge_index[0]
    col = edge_index[1]

    degp = _hist(col)
    d0 = degp[0].reshape(NP, 1)
    d1 = degp[1].reshape(NP, 1)

    x_flat = _mlp(z, mlp_W1, mlp_b1.reshape(1, MLPH), mlp_W2,
                  mlp_b2.reshape(1, N * F))
    x2d = x_flat.reshape(N, F)

    y, dis = _scale(d0, d1, x2d)

    aggp = _agg16(y, row, col)
    y2 = _conv(aggp[0, :N], aggp[1, :N], y, dis,
               conv1_W, conv1_b.reshape(1, HID), conv2_W)

    agg2p = _agg32(y2, row, col)
    out = _final(agg2p[0, :N], agg2p[1, :N], y2, dis,
                 conv2_b.reshape(1, OUTW))
    return out


# speedup vs baseline: 38.5959x; 1.2870x over previous
import functools

import jax
import jax.numpy as jnp
from jax import lax
from jax.experimental import pallas as pl
from jax.experimental.pallas import tpu as pltpu
from jax.experimental.pallas import tpu_sc as plsc

N = 10000
F = 16
HID = 64
OUTW = 32
E = 320000
MLPH = 256

NC, NS = 2, 16
NW = NC * NS
EPW = E // NW
CHUNK = 128
NFULL = EPW // CHUNK
TAIL = EPW - NFULL * CHUNK
PT = 640
NP = NS * PT
DUMMY = N
NCH = NFULL + 1

MLP_BLK = 6400
MLP_STEPS = (N * F) // MLP_BLK

_mesh = plsc.VectorSubcoreMesh(
    core_axis_name="c", subcore_axis_name="s", num_cores=NC, num_subcores=NS
)


def _fill_f32(ref, n, val):
    v = jnp.full((16,), val, jnp.float32)
    for i in range(n // 16):
        ref[pl.ds(16 * i, 16)] = v


@functools.partial(
    pl.kernel,
    out_type=jax.ShapeDtypeStruct((NC, NP), jnp.float32),
    mesh=_mesh,
    scratch_types=[
        pltpu.VMEM((NCH, CHUNK), jnp.int32),
        pltpu.VMEM((CHUNK,), jnp.float32),
        pltpu.VMEM((PT,), jnp.float32),
        pltpu.VMEM_SHARED((NP,), jnp.float32),
        pltpu.SemaphoreType.DMA,
    ],
    compiler_params=pltpu.CompilerParams(use_tc_tiling_on_sc=False),
)
def _hist(col_hbm, out_hbm, cidx_all, ones_v, zeros_v, acc, sem):
    c = lax.axis_index("c")
    s = lax.axis_index("s")
    wid = c * NS + s
    base = wid * EPW
    descs = []
    for k in range(NFULL):
        descs.append(
            pltpu.async_copy(
                col_hbm.at[pl.ds(base + k * CHUNK, CHUNK)], cidx_all.at[k], sem
            )
        )
    descs.append(
        pltpu.async_copy(
            col_hbm.at[pl.ds(base + NFULL * CHUNK, TAIL)],
            cidx_all.at[NFULL, pl.ds(0, TAIL)],
            sem,
        )
    )
    _fill_f32(ones_v, CHUNK, 1.0)
    _fill_f32(zeros_v, PT, 0.0)
    pltpu.sync_copy(zeros_v, acc.at[pl.ds(s * PT, PT)])
    for d in descs:
        d.wait()
    pad = jnp.full((16,), DUMMY, jnp.int32)
    for i in range(TAIL // 16, CHUNK // 16):
        cidx_all[NFULL, pl.ds(16 * i, 16)] = pad
    plsc.subcore_barrier()

    def body(i, carry):
        pltpu.sync_copy(ones_v, acc.at[cidx_all.at[i]], add=True)
        return carry

    lax.fori_loop(0, NCH, body, 0)
    plsc.subcore_barrier()
    pltpu.sync_copy(acc.at[pl.ds(s * PT, PT)], out_hbm.at[c, pl.ds(s * PT, PT)])


def _make_agg(w):
    @functools.partial(
        pl.kernel,
        out_type=jax.ShapeDtypeStruct((NC, NP, w), jnp.float32),
        mesh=_mesh,
        scratch_types=[
            pltpu.VMEM((NCH, CHUNK), jnp.int32),
            pltpu.VMEM((NCH, CHUNK), jnp.int32),
            pltpu.VMEM((4, CHUNK, w), jnp.float32),
            pltpu.VMEM_SHARED((NP, w), jnp.float32),
            pltpu.VMEM_SHARED((NP, w), jnp.float32),
            pltpu.SemaphoreType.DMA,
            pltpu.SemaphoreType.DMA,
        ],
        compiler_params=pltpu.CompilerParams(use_tc_tiling_on_sc=False),
    )
    def agg(y_hbm, row_hbm, col_hbm, out_hbm, ridx_all, cidx_all, rows, acc,
            ytab, semi, semg):
        c = lax.axis_index("c")
        s = lax.axis_index("s")
        wid = c * NS + s
        base = wid * EPW
        descs = []
        for k in range(NFULL):
            off = base + k * CHUNK
            descs.append(
                pltpu.async_copy(row_hbm.at[pl.ds(off, CHUNK)], ridx_all.at[k], semi)
            )
            descs.append(
                pltpu.async_copy(col_hbm.at[pl.ds(off, CHUNK)], cidx_all.at[k], semi)
            )
        off = base + NFULL * CHUNK
        descs.append(
            pltpu.async_copy(
                row_hbm.at[pl.ds(off, TAIL)], ridx_all.at[NFULL, pl.ds(0, TAIL)], semi
            )
        )
        descs.append(
            pltpu.async_copy(
                col_hbm.at[pl.ds(off, TAIL)], cidx_all.at[NFULL, pl.ds(0, TAIL)], semi
            )
        )
        pltpu.sync_copy(
            y_hbm.at[pl.ds(s * PT, PT)], ytab.at[pl.ds(s * PT, PT)]
        )
        zv = jnp.zeros((16,), jnp.float32)
        for r in range(CHUNK):
            for j in range(w // 16):
                rows[0, r, pl.ds(16 * j, 16)] = zv
        for k in range(PT // CHUNK):
            pltpu.sync_copy(
                rows.at[0], acc.at[pl.ds(s * PT + k * CHUNK, CHUNK)]
            )
        for d in descs:
            d.wait()
        zpad = jnp.zeros((16,), jnp.int32)
        dpad = jnp.full((16,), DUMMY, jnp.int32)
        for i in range(TAIL // 16, CHUNK // 16):
            ridx_all[NFULL, pl.ds(16 * i, 16)] = zpad
            cidx_all[NFULL, pl.ds(16 * i, 16)] = dpad
        plsc.subcore_barrier()

        for k in range(3):
            pltpu.async_copy(ytab.at[ridx_all.at[k]], rows.at[k], semg)

        def body(i, carry):
            p = lax.rem(i, 4)
            pltpu.make_async_copy(
                ytab.at[ridx_all.at[i]], rows.at[p], semg
            ).wait()

            @pl.when(i + 3 < NCH)
            def _():
                pltpu.async_copy(
                    ytab.at[ridx_all.at[i + 3]], rows.at[lax.rem(i + 3, 4)], semg
                )

            pltpu.sync_copy(rows.at[p], acc.at[cidx_all.at[i]], add=True)
            return carry

        lax.fori_loop(0, NCH, body, 0)
        plsc.subcore_barrier()
        pltpu.sync_copy(
            acc.at[pl.ds(s * PT, PT)], out_hbm.at[c, pl.ds(s * PT, PT)]
        )

    return agg


_agg16 = _make_agg(F)
_agg32 = _make_agg(OUTW)


def _mlp_body(z_ref, w1_ref, b1_ref, w2_ref, b2_ref, o_ref):
    h1 = jnp.dot(z_ref[...], w1_ref[...], preferred_element_type=jnp.float32)
    h1 = jnp.maximum(h1 + b1_ref[...], 0.0)
    h2 = jnp.dot(h1, w2_ref[...], preferred_element_type=jnp.float32)
    o_ref[...] = jnp.maximum(h2 + b2_ref[...], 0.0)


def _mlp(z, w1, b1, w2, b2):
    return pl.pallas_call(
        _mlp_body,
        grid=(MLP_STEPS,),
        in_specs=[
            pl.BlockSpec((1, F), lambda i: (0, 0)),
            pl.BlockSpec((F, MLPH), lambda i: (0, 0)),
            pl.BlockSpec((1, MLPH), lambda i: (0, 0)),
            pl.BlockSpec((MLPH, MLP_BLK), lambda i: (0, i)),
            pl.BlockSpec((1, MLP_BLK), lambda i: (0, i)),
        ],
        out_specs=pl.BlockSpec((1, MLP_BLK), lambda i: (0, i)),
        out_shape=jax.ShapeDtypeStruct((1, N * F), jnp.float32),
    )(z, w1, b1, w2, b2)


def _scale_body(d0_ref, d1_ref, x_ref, y_ref, dis_ref):
    deg = d0_ref[...] + d1_ref[...] + 1.0
    dis = lax.rsqrt(deg)
    dis10 = lax.slice(dis, (0, 0), (N, 1))
    dis_ref[...] = dis10
    y_ref[pl.ds(0, N), :] = dis10 * x_ref[...]
    y_ref[pl.ds(N, NP - N), :] = jnp.zeros((NP - N, F), jnp.float32)


def _scale(d0, d1, x2d):
    return pl.pallas_call(
        _scale_body,
        grid=(1,),
        in_specs=[
            pl.BlockSpec((NP, 1), lambda i: (0, 0)),
            pl.BlockSpec((NP, 1), lambda i: (0, 0)),
            pl.BlockSpec((N, F), lambda i: (0, 0)),
        ],
        out_specs=[
            pl.BlockSpec((NP, F), lambda i: (0, 0)),
            pl.BlockSpec((N, 1), lambda i: (0, 0)),
        ],
        out_shape=[
            jax.ShapeDtypeStruct((NP, F), jnp.float32),
            jax.ShapeDtypeStruct((N, 1), jnp.float32),
        ],
    )(d0, d1, x2d)


def _conv_body(agg_ref, y_ref, dis_ref, w1c_ref, b1c_ref, w2c_ref, y2_ref):
    dis = dis_ref[...]
    a = agg_ref[...]
    s1 = dis * (a[0] + a[1] + y_ref[...])
    out1 = jnp.dot(s1, w1c_ref[...], preferred_element_type=jnp.float32)
    out1 = jnp.maximum(out1 + b1c_ref[...], 0.0)
    y2 = jnp.dot(out1, w2c_ref[...], preferred_element_type=jnp.float32)
    y2_ref[pl.ds(0, N), :] = dis * y2
    y2_ref[pl.ds(N, NP - N), :] = jnp.zeros((NP - N, OUTW), jnp.float32)


def _conv(aggp, y, dis, w1c, b1c, w2c):
    return pl.pallas_call(
        _conv_body,
        grid=(1,),
        in_specs=[
            pl.BlockSpec((NC, N, F), lambda i: (0, 0, 0)),
            pl.BlockSpec((N, F), lambda i: (0, 0)),
            pl.BlockSpec((N, 1), lambda i: (0, 0)),
            pl.BlockSpec((F, HID), lambda i: (0, 0)),
            pl.BlockSpec((1, HID), lambda i: (0, 0)),
            pl.BlockSpec((HID, OUTW), lambda i: (0, 0)),
        ],
        out_specs=pl.BlockSpec((NP, OUTW), lambda i: (0, 0)),
        out_shape=jax.ShapeDtypeStruct((NP, OUTW), jnp.float32),
    )(aggp, y, dis, w1c, b1c, w2c)


def _final_body(agg_ref, y2_ref, dis_ref, b2c_ref, o_ref):
    b = agg_ref[...]
    o_ref[...] = (
        dis_ref[...] * (b[0] + b[1] + y2_ref[...]) + b2c_ref[...]
    )


def _final(agg2p, y2, dis, b2c):
    return pl.pallas_call(
        _final_body,
        grid=(1,),
        in_specs=[
            pl.BlockSpec((NC, N, OUTW), lambda i: (0, 0, 0)),
            pl.BlockSpec((N, OUTW), lambda i: (0, 0)),
            pl.BlockSpec((N, 1), lambda i: (0, 0)),
            pl.BlockSpec((1, OUTW), lambda i: (0, 0)),
        ],
        out_specs=pl.BlockSpec((N, OUTW), lambda i: (0, 0)),
        out_shape=jax.ShapeDtypeStruct((N, OUTW), jnp.float32),
    )(agg2p, y2, dis, b2c)


def kernel(z, edge_attr, mlp_W1, mlp_b1, mlp_W2, mlp_b2,
           conv1_W, conv1_b, conv2_W, conv2_b, edge_index):
    del edge_attr
    row = edge_index[0]
    col = edge_index[1]

    degp = _hist(col)
    d0 = degp[0].reshape(NP, 1)
    d1 = degp[1].reshape(NP, 1)

    x_flat = _mlp(z, mlp_W1, mlp_b1.reshape(1, MLPH), mlp_W2,
                  mlp_b2.reshape(1, N * F))
    x2d = x_flat.reshape(N, F)

    y, dis = _scale(d0, d1, x2d)

    aggp = _agg16(y, row, col)
    y2 = _conv(aggp, y, dis,
               conv1_W, conv1_b.reshape(1, HID), conv2_W)

    agg2p = _agg32(y2, row, col)
    out = _final(agg2p, y2, dis,
                 conv2_b.reshape(1, OUTW))
    return out


# speedup vs baseline: 42.1990x; 1.0934x over previous
import functools

import jax
import jax.numpy as jnp
from jax import lax
from jax.experimental import pallas as pl
from jax.experimental.pallas import tpu as pltpu
from jax.experimental.pallas import tpu_sc as plsc

N = 10000
F = 16
HID = 64
OUTW = 32
E = 320000
MLPH = 256

NC, NS = 2, 16
NW = NC * NS
EPW = E // NW
CHUNK = 128
NFULL = EPW // CHUNK
TAIL = EPW - NFULL * CHUNK
PT = 640
NP = NS * PT
DUMMY = N
NCH = NFULL + 1

MLP_BLK = 6400
MLP_STEPS = (N * F) // MLP_BLK

_mesh = plsc.VectorSubcoreMesh(
    core_axis_name="c", subcore_axis_name="s", num_cores=NC, num_subcores=NS
)


def _fill_f32(ref, n, val):
    v = jnp.full((16,), val, jnp.float32)
    for i in range(n // 16):
        ref[pl.ds(16 * i, 16)] = v


@functools.partial(
    pl.kernel,
    out_type=jax.ShapeDtypeStruct((NC, NP), jnp.float32),
    mesh=_mesh,
    scratch_types=[
        pltpu.VMEM((NCH, CHUNK), jnp.int32),
        pltpu.VMEM((CHUNK,), jnp.float32),
        pltpu.VMEM((PT,), jnp.float32),
        pltpu.VMEM_SHARED((NP,), jnp.float32),
        pltpu.SemaphoreType.DMA,
    ],
    compiler_params=pltpu.CompilerParams(use_tc_tiling_on_sc=False),
)
def _hist(col_hbm, out_hbm, cidx_all, ones_v, zeros_v, acc, sem):
    c = lax.axis_index("c")
    s = lax.axis_index("s")
    wid = c * NS + s
    base = wid * EPW
    descs = []
    for k in range(NFULL):
        descs.append(
            pltpu.async_copy(
                col_hbm.at[pl.ds(base + k * CHUNK, CHUNK)], cidx_all.at[k], sem
            )
        )
    descs.append(
        pltpu.async_copy(
            col_hbm.at[pl.ds(base + NFULL * CHUNK, TAIL)],
            cidx_all.at[NFULL, pl.ds(0, TAIL)],
            sem,
        )
    )
    _fill_f32(ones_v, CHUNK, 1.0)
    _fill_f32(zeros_v, PT, 0.0)
    pltpu.sync_copy(zeros_v, acc.at[pl.ds(s * PT, PT)])
    for d in descs:
        d.wait()
    pad = jnp.full((16,), DUMMY, jnp.int32)
    for i in range(TAIL // 16, CHUNK // 16):
        cidx_all[NFULL, pl.ds(16 * i, 16)] = pad
    plsc.subcore_barrier()

    def body(i, carry):
        pltpu.sync_copy(ones_v, acc.at[cidx_all.at[i]], add=True)
        return carry

    lax.fori_loop(0, NCH, body, 0)
    plsc.subcore_barrier()
    pltpu.sync_copy(acc.at[pl.ds(s * PT, PT)], out_hbm.at[c, pl.ds(s * PT, PT)])


def _make_agg(w):
    @functools.partial(
        pl.kernel,
        out_type=jax.ShapeDtypeStruct((NC, NP, w), jnp.float32),
        mesh=_mesh,
        scratch_types=[
            pltpu.VMEM((NCH, CHUNK), jnp.int32),
            pltpu.VMEM((NCH, CHUNK), jnp.int32),
            pltpu.VMEM((4, CHUNK, w), jnp.float32),
            pltpu.VMEM_SHARED((NP, w), jnp.float32),
            pltpu.VMEM_SHARED((NP, w), jnp.float32),
            pltpu.SemaphoreType.DMA,
            pltpu.SemaphoreType.DMA,
        ],
        compiler_params=pltpu.CompilerParams(use_tc_tiling_on_sc=False),
    )
    def agg(y_hbm, row_hbm, col_hbm, out_hbm, ridx_all, cidx_all, rows, acc,
            ytab, semi, semg):
        c = lax.axis_index("c")
        s = lax.axis_index("s")
        wid = c * NS + s
        base = wid * EPW
        descs = []
        for k in range(NFULL):
            off = base + k * CHUNK
            descs.append(
                pltpu.async_copy(row_hbm.at[pl.ds(off, CHUNK)], ridx_all.at[k], semi)
            )
            descs.append(
                pltpu.async_copy(col_hbm.at[pl.ds(off, CHUNK)], cidx_all.at[k], semi)
            )
        off = base + NFULL * CHUNK
        descs.append(
            pltpu.async_copy(
                row_hbm.at[pl.ds(off, TAIL)], ridx_all.at[NFULL, pl.ds(0, TAIL)], semi
            )
        )
        descs.append(
            pltpu.async_copy(
                col_hbm.at[pl.ds(off, TAIL)], cidx_all.at[NFULL, pl.ds(0, TAIL)], semi
            )
        )
        pltpu.sync_copy(
            y_hbm.at[pl.ds(s * PT, PT)], ytab.at[pl.ds(s * PT, PT)]
        )
        zv = jnp.zeros((16,), jnp.float32)
        for r in range(CHUNK):
            for j in range(w // 16):
                rows[0, r, pl.ds(16 * j, 16)] = zv
        for k in range(PT // CHUNK):
            pltpu.sync_copy(
                rows.at[0], acc.at[pl.ds(s * PT + k * CHUNK, CHUNK)]
            )
        for d in descs:
            d.wait()
        zpad = jnp.zeros((16,), jnp.int32)
        dpad = jnp.full((16,), DUMMY, jnp.int32)
        for i in range(TAIL // 16, CHUNK // 16):
            ridx_all[NFULL, pl.ds(16 * i, 16)] = zpad
            cidx_all[NFULL, pl.ds(16 * i, 16)] = dpad
        plsc.subcore_barrier()

        for k in range(3):
            pltpu.async_copy(ytab.at[ridx_all.at[k]], rows.at[k], semg)

        def body(i, carry):
            p = lax.rem(i, 4)
            pltpu.make_async_copy(
                ytab.at[ridx_all.at[i]], rows.at[p], semg
            ).wait()

            @pl.when(i + 3 < NCH)
            def _():
                pltpu.async_copy(
                    ytab.at[ridx_all.at[i + 3]], rows.at[lax.rem(i + 3, 4)], semg
                )

            pltpu.sync_copy(rows.at[p], acc.at[cidx_all.at[i]], add=True)
            return carry

        lax.fori_loop(0, NCH, body, 0)
        plsc.subcore_barrier()
        pltpu.sync_copy(
            acc.at[pl.ds(s * PT, PT)], out_hbm.at[c, pl.ds(s * PT, PT)]
        )

    return agg


_agg32 = _make_agg(OUTW)

_RSQRT_MAGIC = 0x5F3759DF


def _rsqrt16(d):
    magic = jnp.full((16,), _RSQRT_MAGIC, jnp.int32)
    one = jnp.full((16,), 1, jnp.int32)
    c15 = jnp.full((16,), 1.5, jnp.float32)
    ch = jnp.full((16,), 0.5, jnp.float32)
    i = lax.bitcast_convert_type(d, jnp.int32)
    i = magic - lax.shift_right_arithmetic(i, one)
    yv = lax.bitcast_convert_type(i, jnp.float32)
    for _ in range(3):
        yv = yv * (c15 - ch * d * yv * yv)
    return yv


@functools.partial(
    pl.kernel,
    out_type=(
        jax.ShapeDtypeStruct((NC, NP, F), jnp.float32),
        jax.ShapeDtypeStruct((NP, F), jnp.float32),
        jax.ShapeDtypeStruct((NP,), jnp.float32),
    ),
    mesh=_mesh,
    scratch_types=[
        pltpu.VMEM((NCH, CHUNK), jnp.int32),
        pltpu.VMEM((NCH, CHUNK), jnp.int32),
        pltpu.VMEM((4, CHUNK, F), jnp.float32),
        pltpu.VMEM((PT, F), jnp.float32),
        pltpu.VMEM((PT,), jnp.float32),
        pltpu.VMEM((PT,), jnp.float32),
        pltpu.VMEM_SHARED((NP, F), jnp.float32),
        pltpu.VMEM_SHARED((NP, F), jnp.float32),
        pltpu.SemaphoreType.DMA,
        pltpu.SemaphoreType.DMA,
    ],
    compiler_params=pltpu.CompilerParams(use_tc_tiling_on_sc=False),
)
def _scale_agg16(x_hbm, deg_hbm, row_hbm, col_hbm,
                 agg_out, y_out, dis_out,
                 ridx_all, cidx_all, rows, xbuf, d0buf, disbuf,
                 acc, ytab, semi, semg):
    c = lax.axis_index("c")
    s = lax.axis_index("s")
    wid = c * NS + s
    base = wid * EPW
    descs = []
    for k in range(NFULL):
        off = base + k * CHUNK
        descs.append(
            pltpu.async_copy(row_hbm.at[pl.ds(off, CHUNK)], ridx_all.at[k], semi)
        )
        descs.append(
            pltpu.async_copy(col_hbm.at[pl.ds(off, CHUNK)], cidx_all.at[k], semi)
        )
    off = base + NFULL * CHUNK
    descs.append(
        pltpu.async_copy(
            row_hbm.at[pl.ds(off, TAIL)], ridx_all.at[NFULL, pl.ds(0, TAIL)], semi
        )
    )
    descs.append(
        pltpu.async_copy(
            col_hbm.at[pl.ds(off, TAIL)], cidx_all.at[NFULL, pl.ds(0, TAIL)], semi
        )
    )
    pltpu.sync_copy(x_hbm.at[pl.ds(s * PT, PT)], xbuf)
    pltpu.sync_copy(deg_hbm.at[0, pl.ds(s * PT, PT)], d0buf)
    pltpu.sync_copy(deg_hbm.at[1, pl.ds(s * PT, PT)], disbuf)
    for g in range(PT // 16):
        dv = d0buf[pl.ds(16 * g, 16)] + disbuf[pl.ds(16 * g, 16)] + 1.0
        disbuf[pl.ds(16 * g, 16)] = _rsqrt16(dv)

    for g in range(PT // 16):
        dv = disbuf[pl.ds(16 * g, 16)]
        for r in range(16):
            splat = jnp.take(dv, jnp.full((16,), r, jnp.int32))
            j = 16 * g + r
            xbuf[j, :] = xbuf[j, :] * splat
    pltpu.sync_copy(xbuf, ytab.at[pl.ds(s * PT, PT)])

    @pl.when(c == 0)
    def _():
        pltpu.sync_copy(xbuf, y_out.at[pl.ds(s * PT, PT)])
        pltpu.sync_copy(disbuf, dis_out.at[pl.ds(s * PT, PT)])

    zv = jnp.zeros((16,), jnp.float32)
    for r in range(CHUNK):
        rows[0, r, :] = zv
    for k in range(PT // CHUNK):
        pltpu.sync_copy(rows.at[0], acc.at[pl.ds(s * PT + k * CHUNK, CHUNK)])
    for d in descs:
        d.wait()
    zpad = jnp.zeros((16,), jnp.int32)
    dpad = jnp.full((16,), DUMMY, jnp.int32)
    for i in range(TAIL // 16, CHUNK // 16):
        ridx_all[NFULL, pl.ds(16 * i, 16)] = zpad
        cidx_all[NFULL, pl.ds(16 * i, 16)] = dpad
    plsc.subcore_barrier()

    for k in range(3):
        pltpu.async_copy(ytab.at[ridx_all.at[k]], rows.at[k], semg)

    def body(i, carry):
        p = lax.rem(i, 4)
        pltpu.make_async_copy(
            ytab.at[ridx_all.at[i]], rows.at[p], semg
        ).wait()

        @pl.when(i + 3 < NCH)
        def _():
            pltpu.async_copy(
                ytab.at[ridx_all.at[i + 3]], rows.at[lax.rem(i + 3, 4)], semg
            )

        pltpu.sync_copy(rows.at[p], acc.at[cidx_all.at[i]], add=True)
        return carry

    lax.fori_loop(0, NCH, body, 0)
    plsc.subcore_barrier()
    pltpu.sync_copy(
        acc.at[pl.ds(s * PT, PT)], agg_out.at[c, pl.ds(s * PT, PT)]
    )


def _mlp_body(z_ref, w1_ref, b1_ref, w2_ref, b2_ref, o_ref):
    h1 = jnp.dot(z_ref[...], w1_ref[...], preferred_element_type=jnp.float32)
    h1 = jnp.maximum(h1 + b1_ref[...], 0.0)
    h2 = jnp.dot(h1, w2_ref[...], preferred_element_type=jnp.float32)
    o_ref[...] = jnp.maximum(h2 + b2_ref[...], 0.0)


def _mlp(z, w1, b1, w2, b2):
    return pl.pallas_call(
        _mlp_body,
        grid=(MLP_STEPS,),
        in_specs=[
            pl.BlockSpec((1, F), lambda i: (0, 0)),
            pl.BlockSpec((F, MLPH), lambda i: (0, 0)),
            pl.BlockSpec((1, MLPH), lambda i: (0, 0)),
            pl.BlockSpec((MLPH, MLP_BLK), lambda i: (0, i)),
            pl.BlockSpec((1, MLP_BLK), lambda i: (0, i)),
        ],
        out_specs=pl.BlockSpec((1, MLP_BLK), lambda i: (0, i)),
        out_shape=jax.ShapeDtypeStruct((1, NP * F), jnp.float32),
    )(z, w1, b1, w2, b2)


def _scale_body(d0_ref, d1_ref, x_ref, y_ref, dis_ref):
    deg = d0_ref[...] + d1_ref[...] + 1.0
    dis = lax.rsqrt(deg)
    dis10 = lax.slice(dis, (0, 0), (N, 1))
    dis_ref[...] = dis10
    y_ref[pl.ds(0, N), :] = dis10 * x_ref[...]
    y_ref[pl.ds(N, NP - N), :] = jnp.zeros((NP - N, F), jnp.float32)


def _scale(d0, d1, x2d):
    return pl.pallas_call(
        _scale_body,
        grid=(1,),
        in_specs=[
            pl.BlockSpec((NP, 1), lambda i: (0, 0)),
            pl.BlockSpec((NP, 1), lambda i: (0, 0)),
            pl.BlockSpec((N, F), lambda i: (0, 0)),
        ],
        out_specs=[
            pl.BlockSpec((NP, F), lambda i: (0, 0)),
            pl.BlockSpec((N, 1), lambda i: (0, 0)),
        ],
        out_shape=[
            jax.ShapeDtypeStruct((NP, F), jnp.float32),
            jax.ShapeDtypeStruct((N, 1), jnp.float32),
        ],
    )(d0, d1, x2d)


def _conv_body(agg_ref, y_ref, dis_ref, w1c_ref, b1c_ref, w2c_ref, y2_ref):
    dis = dis_ref[...]
    a = agg_ref[...]
    s1 = dis * (a[0] + a[1] + y_ref[...])
    out1 = jnp.dot(s1, w1c_ref[...], preferred_element_type=jnp.float32)
    out1 = jnp.maximum(out1 + b1c_ref[...], 0.0)
    y2 = jnp.dot(out1, w2c_ref[...], preferred_element_type=jnp.float32)
    y2_ref[pl.ds(0, N), :] = dis * y2
    y2_ref[pl.ds(N, NP - N), :] = jnp.zeros((NP - N, OUTW), jnp.float32)


def _conv(aggp, y, dis, w1c, b1c, w2c):
    return pl.pallas_call(
        _conv_body,
        grid=(1,),
        in_specs=[
            pl.BlockSpec((NC, N, F), lambda i: (0, 0, 0)),
            pl.BlockSpec((N, F), lambda i: (0, 0)),
            pl.BlockSpec((N, 1), lambda i: (0, 0)),
            pl.BlockSpec((F, HID), lambda i: (0, 0)),
            pl.BlockSpec((1, HID), lambda i: (0, 0)),
            pl.BlockSpec((HID, OUTW), lambda i: (0, 0)),
        ],
        out_specs=pl.BlockSpec((NP, OUTW), lambda i: (0, 0)),
        out_shape=jax.ShapeDtypeStruct((NP, OUTW), jnp.float32),
    )(aggp, y, dis, w1c, b1c, w2c)


def _final_body(agg_ref, y2_ref, dis_ref, b2c_ref, o_ref):
    b = agg_ref[...]
    o_ref[...] = (
        dis_ref[...] * (b[0] + b[1] + y2_ref[...]) + b2c_ref[...]
    )


def _final(agg2p, y2, dis, b2c):
    return pl.pallas_call(
        _final_body,
        grid=(1,),
        in_specs=[
            pl.BlockSpec((NC, N, OUTW), lambda i: (0, 0, 0)),
            pl.BlockSpec((N, OUTW), lambda i: (0, 0)),
            pl.BlockSpec((N, 1), lambda i: (0, 0)),
            pl.BlockSpec((1, OUTW), lambda i: (0, 0)),
        ],
        out_specs=pl.BlockSpec((N, OUTW), lambda i: (0, 0)),
        out_shape=jax.ShapeDtypeStruct((N, OUTW), jnp.float32),
    )(agg2p, y2, dis, b2c)


def kernel(z, edge_attr, mlp_W1, mlp_b1, mlp_W2, mlp_b2,
           conv1_W, conv1_b, conv2_W, conv2_b, edge_index):
    del edge_attr
    row = edge_index[0]
    col = edge_index[1]

    degp = _hist(col)

    x_flat = _mlp(z, mlp_W1, mlp_b1.reshape(1, MLPH), mlp_W2,
                  mlp_b2.reshape(1, N * F))
    x2d = x_flat.reshape(NP, F)

    aggp, y, dis_v = _scale_agg16(x2d, degp, row, col)
    dis = dis_v[:N].reshape(N, 1)
    y2 = _conv(aggp, y, dis,
               conv1_W, conv1_b.reshape(1, HID), conv2_W)

    agg2p = _agg32(y2, row, col)
    out = _final(agg2p, y2, dis,
                 conv2_b.reshape(1, OUTW))
    return out


# speedup vs baseline: 42.7339x; 1.0127x over previous
import functools

import jax
import jax.numpy as jnp
from jax import lax
from jax.experimental import pallas as pl
from jax.experimental.pallas import tpu as pltpu
from jax.experimental.pallas import tpu_sc as plsc

N = 10000
F = 16
HID = 64
OUTW = 32
E = 320000
MLPH = 256

NC, NS = 2, 16
NW = NC * NS
EPW = E // NW
CHUNK = 128
NFULL = EPW // CHUNK
TAIL = EPW - NFULL * CHUNK
PT = 640
NP = NS * PT
DUMMY = N
NCH = NFULL + 1

MLP_BLK = 8192
MLP_STEPS = (NP * F) // MLP_BLK

_mesh = plsc.VectorSubcoreMesh(
    core_axis_name="c", subcore_axis_name="s", num_cores=NC, num_subcores=NS
)


def _fill_f32(ref, n, val):
    v = jnp.full((16,), val, jnp.float32)
    for i in range(n // 16):
        ref[pl.ds(16 * i, 16)] = v


@functools.partial(
    pl.kernel,
    out_type=jax.ShapeDtypeStruct((NC, NP), jnp.float32),
    mesh=_mesh,
    scratch_types=[
        pltpu.VMEM((NCH, CHUNK), jnp.int32),
        pltpu.VMEM((CHUNK,), jnp.float32),
        pltpu.VMEM((PT,), jnp.float32),
        pltpu.VMEM_SHARED((NP,), jnp.float32),
        pltpu.SemaphoreType.DMA,
    ],
    compiler_params=pltpu.CompilerParams(use_tc_tiling_on_sc=False),
)
def _hist(col_hbm, out_hbm, cidx_all, ones_v, zeros_v, acc, sem):
    c = lax.axis_index("c")
    s = lax.axis_index("s")
    wid = c * NS + s
    base = wid * EPW
    descs = []
    for k in range(NFULL):
        descs.append(
            pltpu.async_copy(
                col_hbm.at[pl.ds(base + k * CHUNK, CHUNK)], cidx_all.at[k], sem
            )
        )
    descs.append(
        pltpu.async_copy(
            col_hbm.at[pl.ds(base + NFULL * CHUNK, TAIL)],
            cidx_all.at[NFULL, pl.ds(0, TAIL)],
            sem,
        )
    )
    _fill_f32(ones_v, CHUNK, 1.0)
    _fill_f32(zeros_v, PT, 0.0)
    pltpu.sync_copy(zeros_v, acc.at[pl.ds(s * PT, PT)])
    for d in descs:
        d.wait()
    pad = jnp.full((16,), DUMMY, jnp.int32)
    for i in range(TAIL // 16, CHUNK // 16):
        cidx_all[NFULL, pl.ds(16 * i, 16)] = pad
    plsc.subcore_barrier()

    def body(i, carry):
        pltpu.sync_copy(ones_v, acc.at[cidx_all.at[i]], add=True)
        return carry

    lax.fori_loop(0, NCH, body, 0)
    plsc.subcore_barrier()
    pltpu.sync_copy(acc.at[pl.ds(s * PT, PT)], out_hbm.at[c, pl.ds(s * PT, PT)])


def _make_agg(w):
    @functools.partial(
        pl.kernel,
        out_type=jax.ShapeDtypeStruct((NC, NP, w), jnp.float32),
        mesh=_mesh,
        scratch_types=[
            pltpu.VMEM((NCH, CHUNK), jnp.int32),
            pltpu.VMEM((NCH, CHUNK), jnp.int32),
            pltpu.VMEM((4, CHUNK, w), jnp.float32),
            pltpu.VMEM_SHARED((NP, w), jnp.float32),
            pltpu.VMEM_SHARED((NP, w), jnp.float32),
            pltpu.SemaphoreType.DMA,
            pltpu.SemaphoreType.DMA,
        ],
        compiler_params=pltpu.CompilerParams(use_tc_tiling_on_sc=False),
    )
    def agg(y_hbm, row_hbm, col_hbm, out_hbm, ridx_all, cidx_all, rows, acc,
            ytab, semi, semg):
        c = lax.axis_index("c")
        s = lax.axis_index("s")
        wid = c * NS + s
        base = wid * EPW
        descs = []
        for k in range(NFULL):
            off = base + k * CHUNK
            descs.append(
                pltpu.async_copy(row_hbm.at[pl.ds(off, CHUNK)], ridx_all.at[k], semi)
            )
            descs.append(
                pltpu.async_copy(col_hbm.at[pl.ds(off, CHUNK)], cidx_all.at[k], semi)
            )
        off = base + NFULL * CHUNK
        descs.append(
            pltpu.async_copy(
                row_hbm.at[pl.ds(off, TAIL)], ridx_all.at[NFULL, pl.ds(0, TAIL)], semi
            )
        )
        descs.append(
            pltpu.async_copy(
                col_hbm.at[pl.ds(off, TAIL)], cidx_all.at[NFULL, pl.ds(0, TAIL)], semi
            )
        )
        pltpu.sync_copy(
            y_hbm.at[pl.ds(s * PT, PT)], ytab.at[pl.ds(s * PT, PT)]
        )
        zv = jnp.zeros((16,), jnp.float32)
        for r in range(CHUNK):
            for j in range(w // 16):
                rows[0, r, pl.ds(16 * j, 16)] = zv
        for k in range(PT // CHUNK):
            pltpu.sync_copy(
                rows.at[0], acc.at[pl.ds(s * PT + k * CHUNK, CHUNK)]
            )
        for d in descs:
            d.wait()
        zpad = jnp.zeros((16,), jnp.int32)
        dpad = jnp.full((16,), DUMMY, jnp.int32)
        for i in range(TAIL // 16, CHUNK // 16):
            ridx_all[NFULL, pl.ds(16 * i, 16)] = zpad
            cidx_all[NFULL, pl.ds(16 * i, 16)] = dpad
        plsc.subcore_barrier()

        for k in range(3):
            pltpu.async_copy(ytab.at[ridx_all.at[k]], rows.at[k], semg)

        def body(i, carry):
            p = lax.rem(i, 4)
            pltpu.make_async_copy(
                ytab.at[ridx_all.at[i]], rows.at[p], semg
            ).wait()

            @pl.when(i + 3 < NCH)
            def _():
                pltpu.async_copy(
                    ytab.at[ridx_all.at[i + 3]], rows.at[lax.rem(i + 3, 4)], semg
                )

            pltpu.sync_copy(rows.at[p], acc.at[cidx_all.at[i]], add=True)
            return carry

        lax.fori_loop(0, NCH, body, 0)
        plsc.subcore_barrier()
        pltpu.sync_copy(
            acc.at[pl.ds(s * PT, PT)], out_hbm.at[c, pl.ds(s * PT, PT)]
        )

    return agg


_agg32 = _make_agg(OUTW)

_RSQRT_MAGIC = 0x5F3759DF


def _rsqrt16(d):
    magic = jnp.full((16,), _RSQRT_MAGIC, jnp.int32)
    one = jnp.full((16,), 1, jnp.int32)
    c15 = jnp.full((16,), 1.5, jnp.float32)
    ch = jnp.full((16,), 0.5, jnp.float32)
    i = lax.bitcast_convert_type(d, jnp.int32)
    i = magic - lax.shift_right_arithmetic(i, one)
    yv = lax.bitcast_convert_type(i, jnp.float32)
    for _ in range(3):
        yv = yv * (c15 - ch * d * yv * yv)
    return yv


@functools.partial(
    pl.kernel,
    out_type=(
        jax.ShapeDtypeStruct((NC, NP, F), jnp.float32),
        jax.ShapeDtypeStruct((NP, F), jnp.float32),
        jax.ShapeDtypeStruct((NP,), jnp.float32),
    ),
    mesh=_mesh,
    scratch_types=[
        pltpu.VMEM((NCH, CHUNK), jnp.int32),
        pltpu.VMEM((NCH, CHUNK), jnp.int32),
        pltpu.VMEM((4, CHUNK, F), jnp.float32),
        pltpu.VMEM((PT, F), jnp.float32),
        pltpu.VMEM((PT,), jnp.float32),
        pltpu.VMEM((PT,), jnp.float32),
        pltpu.VMEM_SHARED((NP, F), jnp.float32),
        pltpu.VMEM_SHARED((NP, F), jnp.float32),
        pltpu.SemaphoreType.DMA,
        pltpu.SemaphoreType.DMA,
    ],
    compiler_params=pltpu.CompilerParams(use_tc_tiling_on_sc=False),
)
def _scale_agg16(x_hbm, deg_hbm, row_hbm, col_hbm,
                 agg_out, y_out, dis_out,
                 ridx_all, cidx_all, rows, xbuf, d0buf, disbuf,
                 acc, ytab, semi, semg):
    c = lax.axis_index("c")
    s = lax.axis_index("s")
    wid = c * NS + s
    base = wid * EPW
    descs = []
    for k in range(NFULL):
        off = base + k * CHUNK
        descs.append(
            pltpu.async_copy(row_hbm.at[pl.ds(off, CHUNK)], ridx_all.at[k], semi)
        )
        descs.append(
            pltpu.async_copy(col_hbm.at[pl.ds(off, CHUNK)], cidx_all.at[k], semi)
        )
    off = base + NFULL * CHUNK
    descs.append(
        pltpu.async_copy(
            row_hbm.at[pl.ds(off, TAIL)], ridx_all.at[NFULL, pl.ds(0, TAIL)], semi
        )
    )
    descs.append(
        pltpu.async_copy(
            col_hbm.at[pl.ds(off, TAIL)], cidx_all.at[NFULL, pl.ds(0, TAIL)], semi
        )
    )
    pltpu.sync_copy(x_hbm.at[pl.ds(s * PT, PT)], xbuf)
    pltpu.sync_copy(deg_hbm.at[0, pl.ds(s * PT, PT)], d0buf)
    pltpu.sync_copy(deg_hbm.at[1, pl.ds(s * PT, PT)], disbuf)
    for g in range(PT // 16):
        dv = d0buf[pl.ds(16 * g, 16)] + disbuf[pl.ds(16 * g, 16)] + 1.0
        disbuf[pl.ds(16 * g, 16)] = _rsqrt16(dv)

    for g in range(PT // 16):
        dv = disbuf[pl.ds(16 * g, 16)]
        for r in range(16):
            splat = jnp.take(dv, jnp.full((16,), r, jnp.int32))
            j = 16 * g + r
            xbuf[j, :] = xbuf[j, :] * splat
    pltpu.sync_copy(xbuf, ytab.at[pl.ds(s * PT, PT)])

    @pl.when(c == 0)
    def _():
        pltpu.sync_copy(xbuf, y_out.at[pl.ds(s * PT, PT)])
        pltpu.sync_copy(disbuf, dis_out.at[pl.ds(s * PT, PT)])

    zv = jnp.zeros((16,), jnp.float32)
    for r in range(CHUNK):
        rows[0, r, :] = zv
    for k in range(PT // CHUNK):
        pltpu.sync_copy(rows.at[0], acc.at[pl.ds(s * PT + k * CHUNK, CHUNK)])
    for d in descs:
        d.wait()
    zpad = jnp.zeros((16,), jnp.int32)
    dpad = jnp.full((16,), DUMMY, jnp.int32)
    for i in range(TAIL // 16, CHUNK // 16):
        ridx_all[NFULL, pl.ds(16 * i, 16)] = zpad
        cidx_all[NFULL, pl.ds(16 * i, 16)] = dpad
    plsc.subcore_barrier()

    for k in range(3):
        pltpu.async_copy(ytab.at[ridx_all.at[k]], rows.at[k], semg)

    def body(i, carry):
        p = lax.rem(i, 4)
        pltpu.make_async_copy(
            ytab.at[ridx_all.at[i]], rows.at[p], semg
        ).wait()

        @pl.when(i + 3 < NCH)
        def _():
            pltpu.async_copy(
                ytab.at[ridx_all.at[i + 3]], rows.at[lax.rem(i + 3, 4)], semg
            )

        pltpu.sync_copy(rows.at[p], acc.at[cidx_all.at[i]], add=True)
        return carry

    lax.fori_loop(0, NCH, body, 0)
    plsc.subcore_barrier()
    pltpu.sync_copy(
        acc.at[pl.ds(s * PT, PT)], agg_out.at[c, pl.ds(s * PT, PT)]
    )


def _mlp_body(z_ref, w1_ref, b1_ref, w2_ref, b2_ref, o_ref):
    h1 = jnp.dot(z_ref[...], w1_ref[...], preferred_element_type=jnp.float32)
    h1 = jnp.maximum(h1 + b1_ref[...], 0.0)
    h2 = jnp.dot(h1, w2_ref[...], preferred_element_type=jnp.float32)
    x = jnp.maximum(h2 + b2_ref[...], 0.0)
    o_ref[...] = x.reshape(MLP_BLK // 128, 128)


def _mlp(z, w1, b1, w2, b2):
    return pl.pallas_call(
        _mlp_body,
        grid=(MLP_STEPS,),
        in_specs=[
            pl.BlockSpec((1, F), lambda i: (0, 0)),
            pl.BlockSpec((F, MLPH), lambda i: (0, 0)),
            pl.BlockSpec((1, MLPH), lambda i: (0, 0)),
            pl.BlockSpec((MLPH, MLP_BLK), lambda i: (0, i)),
            pl.BlockSpec((1, MLP_BLK), lambda i: (0, i)),
        ],
        out_specs=pl.BlockSpec((MLP_BLK // 128, 128), lambda i: (i, 0)),
        out_shape=jax.ShapeDtypeStruct((NP * F // 128, 128), jnp.float32),
    )(z, w1, b1, w2, b2)


def _scale_body(d0_ref, d1_ref, x_ref, y_ref, dis_ref):
    deg = d0_ref[...] + d1_ref[...] + 1.0
    dis = lax.rsqrt(deg)
    dis10 = lax.slice(dis, (0, 0), (N, 1))
    dis_ref[...] = dis10
    y_ref[pl.ds(0, N), :] = dis10 * x_ref[...]
    y_ref[pl.ds(N, NP - N), :] = jnp.zeros((NP - N, F), jnp.float32)


def _scale(d0, d1, x2d):
    return pl.pallas_call(
        _scale_body,
        grid=(1,),
        in_specs=[
            pl.BlockSpec((NP, 1), lambda i: (0, 0)),
            pl.BlockSpec((NP, 1), lambda i: (0, 0)),
            pl.BlockSpec((N, F), lambda i: (0, 0)),
        ],
        out_specs=[
            pl.BlockSpec((NP, F), lambda i: (0, 0)),
            pl.BlockSpec((N, 1), lambda i: (0, 0)),
        ],
        out_shape=[
            jax.ShapeDtypeStruct((NP, F), jnp.float32),
            jax.ShapeDtypeStruct((N, 1), jnp.float32),
        ],
    )(d0, d1, x2d)


def _conv_body(agg_ref, y_ref, dis_ref, w1c_ref, b1c_ref, w2c_ref, y2_ref):
    dis = dis_ref[...]
    a = agg_ref[...]
    s1 = dis * (a[0] + a[1] + y_ref[...])
    out1 = jnp.dot(s1, w1c_ref[...], preferred_element_type=jnp.float32)
    out1 = jnp.maximum(out1 + b1c_ref[...], 0.0)
    y2 = jnp.dot(out1, w2c_ref[...], preferred_element_type=jnp.float32)
    y2_ref[pl.ds(0, N), :] = dis * y2
    y2_ref[pl.ds(N, NP - N), :] = jnp.zeros((NP - N, OUTW), jnp.float32)


def _conv(aggp, y, dis, w1c, b1c, w2c):
    return pl.pallas_call(
        _conv_body,
        grid=(1,),
        in_specs=[
            pl.BlockSpec((NC, N, F), lambda i: (0, 0, 0)),
            pl.BlockSpec((N, F), lambda i: (0, 0)),
            pl.BlockSpec((N, 1), lambda i: (0, 0)),
            pl.BlockSpec((F, HID), lambda i: (0, 0)),
            pl.BlockSpec((1, HID), lambda i: (0, 0)),
            pl.BlockSpec((HID, OUTW), lambda i: (0, 0)),
        ],
        out_specs=pl.BlockSpec((NP, OUTW), lambda i: (0, 0)),
        out_shape=jax.ShapeDtypeStruct((NP, OUTW), jnp.float32),
    )(aggp, y, dis, w1c, b1c, w2c)


def _final_body(agg_ref, y2_ref, dis_ref, b2c_ref, o_ref):
    b = agg_ref[...]
    o_ref[...] = (
        dis_ref[...] * (b[0] + b[1] + y2_ref[...]) + b2c_ref[...]
    )


def _final(agg2p, y2, dis, b2c):
    return pl.pallas_call(
        _final_body,
        grid=(1,),
        in_specs=[
            pl.BlockSpec((NC, N, OUTW), lambda i: (0, 0, 0)),
            pl.BlockSpec((N, OUTW), lambda i: (0, 0)),
            pl.BlockSpec((N, 1), lambda i: (0, 0)),
            pl.BlockSpec((1, OUTW), lambda i: (0, 0)),
        ],
        out_specs=pl.BlockSpec((N, OUTW), lambda i: (0, 0)),
        out_shape=jax.ShapeDtypeStruct((N, OUTW), jnp.float32),
    )(agg2p, y2, dis, b2c)


def kernel(z, edge_attr, mlp_W1, mlp_b1, mlp_W2, mlp_b2,
           conv1_W, conv1_b, conv2_W, conv2_b, edge_index):
    del edge_attr
    row = edge_index[0]
    col = edge_index[1]

    degp = _hist(col)

    x_pack = _mlp(z, mlp_W1, mlp_b1.reshape(1, MLPH), mlp_W2,
                  mlp_b2.reshape(1, N * F))
    x2d = x_pack.reshape(NP, F)

    aggp, y, dis_v = _scale_agg16(x2d, degp, row, col)
    dis = dis_v[:N].reshape(N, 1)
    y2 = _conv(aggp, y, dis,
               conv1_W, conv1_b.reshape(1, HID), conv2_W)

    agg2p = _agg32(y2, row, col)
    out = _final(agg2p, y2, dis,
                 conv2_b.reshape(1, OUTW))
    return out


# speedup vs baseline: 42.8806x; 1.0034x over previous
import functools

import jax
import jax.numpy as jnp
from jax import lax
from jax.experimental import pallas as pl
from jax.experimental.pallas import tpu as pltpu
from jax.experimental.pallas import tpu_sc as plsc

N = 10000
F = 16
HID = 64
OUTW = 32
E = 320000
MLPH = 256

NC, NS = 2, 16
NW = NC * NS
EPW = E // NW
CHUNK = 128
NFULL = EPW // CHUNK
TAIL = EPW - NFULL * CHUNK
PT = 640
NP = NS * PT
DUMMY = N
NCH = NFULL + 1

MLP_BLK = 8192
MLP_STEPS = (NP * F) // MLP_BLK

_mesh = plsc.VectorSubcoreMesh(
    core_axis_name="c", subcore_axis_name="s", num_cores=NC, num_subcores=NS
)


def _fill_f32(ref, n, val):
    v = jnp.full((16,), val, jnp.float32)
    for i in range(n // 16):
        ref[pl.ds(16 * i, 16)] = v


@functools.partial(
    pl.kernel,
    out_type=jax.ShapeDtypeStruct((NC, NP), jnp.float32),
    mesh=_mesh,
    scratch_types=[
        pltpu.VMEM((NCH, CHUNK), jnp.int32),
        pltpu.VMEM((CHUNK,), jnp.float32),
        pltpu.VMEM((PT,), jnp.float32),
        pltpu.VMEM_SHARED((NP,), jnp.float32),
        pltpu.SemaphoreType.DMA,
    ],
    compiler_params=pltpu.CompilerParams(use_tc_tiling_on_sc=False),
)
def _hist(col_hbm, out_hbm, cidx_all, ones_v, zeros_v, acc, sem):
    c = lax.axis_index("c")
    s = lax.axis_index("s")
    wid = c * NS + s
    base = wid * EPW
    descs = []
    for k in range(NFULL):
        descs.append(
            pltpu.async_copy(
                col_hbm.at[pl.ds(base + k * CHUNK, CHUNK)], cidx_all.at[k], sem
            )
        )
    descs.append(
        pltpu.async_copy(
            col_hbm.at[pl.ds(base + NFULL * CHUNK, TAIL)],
            cidx_all.at[NFULL, pl.ds(0, TAIL)],
            sem,
        )
    )
    _fill_f32(ones_v, CHUNK, 1.0)
    _fill_f32(zeros_v, PT, 0.0)
    pltpu.sync_copy(zeros_v, acc.at[pl.ds(s * PT, PT)])
    for d in descs:
        d.wait()
    pad = jnp.full((16,), DUMMY, jnp.int32)
    for i in range(TAIL // 16, CHUNK // 16):
        cidx_all[NFULL, pl.ds(16 * i, 16)] = pad
    plsc.subcore_barrier()

    def body(i, carry):
        pltpu.sync_copy(ones_v, acc.at[cidx_all.at[i]], add=True)
        return carry

    lax.fori_loop(0, NCH, body, 0)
    plsc.subcore_barrier()
    pltpu.sync_copy(acc.at[pl.ds(s * PT, PT)], out_hbm.at[c, pl.ds(s * PT, PT)])


def _make_agg(w):
    @functools.partial(
        pl.kernel,
        out_type=jax.ShapeDtypeStruct((NC, NP, w), jnp.float32),
        mesh=_mesh,
        scratch_types=[
            pltpu.VMEM((NCH, CHUNK), jnp.int32),
            pltpu.VMEM((NCH, CHUNK), jnp.int32),
            pltpu.VMEM((4, CHUNK, w), jnp.float32),
            pltpu.VMEM_SHARED((NP, w), jnp.float32),
            pltpu.VMEM_SHARED((NP, w), jnp.float32),
            pltpu.SemaphoreType.DMA,
            pltpu.SemaphoreType.DMA,
            pltpu.SemaphoreType.DMA,
        ],
        compiler_params=pltpu.CompilerParams(use_tc_tiling_on_sc=False),
    )
    def agg(y_hbm, row_hbm, col_hbm, out_hbm, ridx_all, cidx_all, rows, acc,
            ytab, semi, semg, sems):
        c = lax.axis_index("c")
        s = lax.axis_index("s")
        wid = c * NS + s
        base = wid * EPW
        descs = []
        for k in range(NFULL):
            off = base + k * CHUNK
            descs.append(
                pltpu.async_copy(row_hbm.at[pl.ds(off, CHUNK)], ridx_all.at[k], semi)
            )
            descs.append(
                pltpu.async_copy(col_hbm.at[pl.ds(off, CHUNK)], cidx_all.at[k], semi)
            )
        off = base + NFULL * CHUNK
        descs.append(
            pltpu.async_copy(
                row_hbm.at[pl.ds(off, TAIL)], ridx_all.at[NFULL, pl.ds(0, TAIL)], semi
            )
        )
        descs.append(
            pltpu.async_copy(
                col_hbm.at[pl.ds(off, TAIL)], cidx_all.at[NFULL, pl.ds(0, TAIL)], semi
            )
        )
        pltpu.sync_copy(
            y_hbm.at[pl.ds(s * PT, PT)], ytab.at[pl.ds(s * PT, PT)]
        )
        zv = jnp.zeros((16,), jnp.float32)
        for r in range(CHUNK):
            for j in range(w // 16):
                rows[0, r, pl.ds(16 * j, 16)] = zv
        for k in range(PT // CHUNK):
            pltpu.sync_copy(
                rows.at[0], acc.at[pl.ds(s * PT + k * CHUNK, CHUNK)]
            )
        for d in descs:
            d.wait()
        zpad = jnp.zeros((16,), jnp.int32)
        dpad = jnp.full((16,), DUMMY, jnp.int32)
        for i in range(TAIL // 16, CHUNK // 16):
            ridx_all[NFULL, pl.ds(16 * i, 16)] = zpad
            cidx_all[NFULL, pl.ds(16 * i, 16)] = dpad
        plsc.subcore_barrier()

        for k in range(3):
            pltpu.async_copy(ytab.at[ridx_all.at[k]], rows.at[k], semg)

        def body(i, carry):
            p = lax.rem(i, 4)
            pltpu.make_async_copy(
                ytab.at[ridx_all.at[i]], rows.at[p], semg
            ).wait()

            @pl.when(i >= 1)
            def _():
                pltpu.make_async_copy(
                    rows.at[lax.rem(i + 3, 4)], acc.at[cidx_all.at[i]], sems
                ).wait()

            pltpu.async_copy(rows.at[p], acc.at[cidx_all.at[i]], sems, add=True)

            @pl.when(i + 3 < NCH)
            def _():
                pltpu.async_copy(
                    ytab.at[ridx_all.at[i + 3]], rows.at[lax.rem(i + 3, 4)], semg
                )

            return carry

        lax.fori_loop(0, NCH, body, 0)
        pltpu.make_async_copy(
            rows.at[0], acc.at[cidx_all.at[0]], sems
        ).wait()
        plsc.subcore_barrier()
        pltpu.sync_copy(
            acc.at[pl.ds(s * PT, PT)], out_hbm.at[c, pl.ds(s * PT, PT)]
        )

    return agg


_agg32 = _make_agg(OUTW)

_RSQRT_MAGIC = 0x5F3759DF


def _rsqrt16(d):
    magic = jnp.full((16,), _RSQRT_MAGIC, jnp.int32)
    one = jnp.full((16,), 1, jnp.int32)
    c15 = jnp.full((16,), 1.5, jnp.float32)
    ch = jnp.full((16,), 0.5, jnp.float32)
    i = lax.bitcast_convert_type(d, jnp.int32)
    i = magic - lax.shift_right_arithmetic(i, one)
    yv = lax.bitcast_convert_type(i, jnp.float32)
    for _ in range(3):
        yv = yv * (c15 - ch * d * yv * yv)
    return yv


@functools.partial(
    pl.kernel,
    out_type=(
        jax.ShapeDtypeStruct((NC, NP, F), jnp.float32),
        jax.ShapeDtypeStruct((NP, F), jnp.float32),
        jax.ShapeDtypeStruct((NP,), jnp.float32),
    ),
    mesh=_mesh,
    scratch_types=[
        pltpu.VMEM((NCH, CHUNK), jnp.int32),
        pltpu.VMEM((NCH, CHUNK), jnp.int32),
        pltpu.VMEM((4, CHUNK, F), jnp.float32),
        pltpu.VMEM((PT, F), jnp.float32),
        pltpu.VMEM((PT,), jnp.float32),
        pltpu.VMEM((PT,), jnp.float32),
        pltpu.VMEM_SHARED((NP, F), jnp.float32),
        pltpu.VMEM_SHARED((NP, F), jnp.float32),
        pltpu.SemaphoreType.DMA,
        pltpu.SemaphoreType.DMA,
        pltpu.SemaphoreType.DMA,
    ],
    compiler_params=pltpu.CompilerParams(use_tc_tiling_on_sc=False),
)
def _scale_agg16(x_hbm, deg_hbm, row_hbm, col_hbm,
                 agg_out, y_out, dis_out,
                 ridx_all, cidx_all, rows, xbuf, d0buf, disbuf,
                 acc, ytab, semi, semg, sems):
    c = lax.axis_index("c")
    s = lax.axis_index("s")
    wid = c * NS + s
    base = wid * EPW
    descs = []
    for k in range(NFULL):
        off = base + k * CHUNK
        descs.append(
            pltpu.async_copy(row_hbm.at[pl.ds(off, CHUNK)], ridx_all.at[k], semi)
        )
        descs.append(
            pltpu.async_copy(col_hbm.at[pl.ds(off, CHUNK)], cidx_all.at[k], semi)
        )
    off = base + NFULL * CHUNK
    descs.append(
        pltpu.async_copy(
            row_hbm.at[pl.ds(off, TAIL)], ridx_all.at[NFULL, pl.ds(0, TAIL)], semi
        )
    )
    descs.append(
        pltpu.async_copy(
            col_hbm.at[pl.ds(off, TAIL)], cidx_all.at[NFULL, pl.ds(0, TAIL)], semi
        )
    )
    pltpu.sync_copy(x_hbm.at[pl.ds(s * PT, PT)], xbuf)
    pltpu.sync_copy(deg_hbm.at[0, pl.ds(s * PT, PT)], d0buf)
    pltpu.sync_copy(deg_hbm.at[1, pl.ds(s * PT, PT)], disbuf)
    for g in range(PT // 16):
        dv = d0buf[pl.ds(16 * g, 16)] + disbuf[pl.ds(16 * g, 16)] + 1.0
        disbuf[pl.ds(16 * g, 16)] = _rsqrt16(dv)

    for g in range(PT // 16):
        dv = disbuf[pl.ds(16 * g, 16)]
        for r in range(16):
            splat = jnp.take(dv, jnp.full((16,), r, jnp.int32))
            j = 16 * g + r
            xbuf[j, :] = xbuf[j, :] * splat
    pltpu.sync_copy(xbuf, ytab.at[pl.ds(s * PT, PT)])

    @pl.when(c == 0)
    def _():
        pltpu.sync_copy(xbuf, y_out.at[pl.ds(s * PT, PT)])
        pltpu.sync_copy(disbuf, dis_out.at[pl.ds(s * PT, PT)])

    zv = jnp.zeros((16,), jnp.float32)
    for r in range(CHUNK):
        rows[0, r, :] = zv
    for k in range(PT // CHUNK):
        pltpu.sync_copy(rows.at[0], acc.at[pl.ds(s * PT + k * CHUNK, CHUNK)])
    for d in descs:
        d.wait()
    zpad = jnp.zeros((16,), jnp.int32)
    dpad = jnp.full((16,), DUMMY, jnp.int32)
    for i in range(TAIL // 16, CHUNK // 16):
        ridx_all[NFULL, pl.ds(16 * i, 16)] = zpad
        cidx_all[NFULL, pl.ds(16 * i, 16)] = dpad
    plsc.subcore_barrier()

    for k in range(3):
        pltpu.async_copy(ytab.at[ridx_all.at[k]], rows.at[k], semg)

    def body(i, carry):
        p = lax.rem(i, 4)
        pltpu.make_async_copy(
            ytab.at[ridx_all.at[i]], rows.at[p], semg
        ).wait()

        @pl.when(i >= 1)
        def _():
            pltpu.make_async_copy(
                rows.at[lax.rem(i + 3, 4)], acc.at[cidx_all.at[i]], sems
            ).wait()

        pltpu.async_copy(rows.at[p], acc.at[cidx_all.at[i]], sems, add=True)

        @pl.when(i + 3 < NCH)
        def _():
            pltpu.async_copy(
                ytab.at[ridx_all.at[i + 3]], rows.at[lax.rem(i + 3, 4)], semg
            )

        return carry

    lax.fori_loop(0, NCH, body, 0)
    pltpu.make_async_copy(
        rows.at[0], acc.at[cidx_all.at[0]], sems
    ).wait()
    plsc.subcore_barrier()
    pltpu.sync_copy(
        acc.at[pl.ds(s * PT, PT)], agg_out.at[c, pl.ds(s * PT, PT)]
    )


def _mlp_body(z_ref, w1_ref, b1_ref, w2_ref, b2_ref, o_ref):
    h1 = jnp.dot(z_ref[...], w1_ref[...], preferred_element_type=jnp.float32)
    h1 = jnp.maximum(h1 + b1_ref[...], 0.0)
    h2 = jnp.dot(h1, w2_ref[...], preferred_element_type=jnp.float32)
    x = jnp.maximum(h2 + b2_ref[...], 0.0)
    o_ref[...] = x.reshape(MLP_BLK // 128, 128)


def _mlp(z, w1, b1, w2, b2):
    return pl.pallas_call(
        _mlp_body,
        grid=(MLP_STEPS,),
        in_specs=[
            pl.BlockSpec((1, F), lambda i: (0, 0)),
            pl.BlockSpec((F, MLPH), lambda i: (0, 0)),
            pl.BlockSpec((1, MLPH), lambda i: (0, 0)),
            pl.BlockSpec((MLPH, MLP_BLK), lambda i: (0, i)),
            pl.BlockSpec((1, MLP_BLK), lambda i: (0, i)),
        ],
        out_specs=pl.BlockSpec((MLP_BLK // 128, 128), lambda i: (i, 0)),
        out_shape=jax.ShapeDtypeStruct((NP * F // 128, 128), jnp.float32),
    )(z, w1, b1, w2, b2)


def _scale_body(d0_ref, d1_ref, x_ref, y_ref, dis_ref):
    deg = d0_ref[...] + d1_ref[...] + 1.0
    dis = lax.rsqrt(deg)
    dis10 = lax.slice(dis, (0, 0), (N, 1))
    dis_ref[...] = dis10
    y_ref[pl.ds(0, N), :] = dis10 * x_ref[...]
    y_ref[pl.ds(N, NP - N), :] = jnp.zeros((NP - N, F), jnp.float32)


def _scale(d0, d1, x2d):
    return pl.pallas_call(
        _scale_body,
        grid=(1,),
        in_specs=[
            pl.BlockSpec((NP, 1), lambda i: (0, 0)),
            pl.BlockSpec((NP, 1), lambda i: (0, 0)),
            pl.BlockSpec((N, F), lambda i: (0, 0)),
        ],
        out_specs=[
            pl.BlockSpec((NP, F), lambda i: (0, 0)),
            pl.BlockSpec((N, 1), lambda i: (0, 0)),
        ],
        out_shape=[
            jax.ShapeDtypeStruct((NP, F), jnp.float32),
            jax.ShapeDtypeStruct((N, 1), jnp.float32),
        ],
    )(d0, d1, x2d)


def _conv_body(agg_ref, y_ref, dis_ref, w1c_ref, b1c_ref, w2c_ref, y2_ref):
    dis = dis_ref[...]
    a = agg_ref[...]
    s1 = dis * (a[0] + a[1] + y_ref[...])
    out1 = jnp.dot(s1, w1c_ref[...], preferred_element_type=jnp.float32)
    out1 = jnp.maximum(out1 + b1c_ref[...], 0.0)
    y2 = jnp.dot(out1, w2c_ref[...], preferred_element_type=jnp.float32)
    y2_ref[pl.ds(0, N), :] = dis * y2
    y2_ref[pl.ds(N, NP - N), :] = jnp.zeros((NP - N, OUTW), jnp.float32)


def _conv(aggp, y, dis, w1c, b1c, w2c):
    return pl.pallas_call(
        _conv_body,
        grid=(1,),
        in_specs=[
            pl.BlockSpec((NC, N, F), lambda i: (0, 0, 0)),
            pl.BlockSpec((N, F), lambda i: (0, 0)),
            pl.BlockSpec((N, 1), lambda i: (0, 0)),
            pl.BlockSpec((F, HID), lambda i: (0, 0)),
            pl.BlockSpec((1, HID), lambda i: (0, 0)),
            pl.BlockSpec((HID, OUTW), lambda i: (0, 0)),
        ],
        out_specs=pl.BlockSpec((NP, OUTW), lambda i: (0, 0)),
        out_shape=jax.ShapeDtypeStruct((NP, OUTW), jnp.float32),
    )(aggp, y, dis, w1c, b1c, w2c)


def _final_body(agg_ref, y2_ref, dis_ref, b2c_ref, o_ref):
    b = agg_ref[...]
    o_ref[...] = (
        dis_ref[...] * (b[0] + b[1] + y2_ref[...]) + b2c_ref[...]
    )


def _final(agg2p, y2, dis, b2c):
    return pl.pallas_call(
        _final_body,
        grid=(1,),
        in_specs=[
            pl.BlockSpec((NC, N, OUTW), lambda i: (0, 0, 0)),
            pl.BlockSpec((N, OUTW), lambda i: (0, 0)),
            pl.BlockSpec((N, 1), lambda i: (0, 0)),
            pl.BlockSpec((1, OUTW), lambda i: (0, 0)),
        ],
        out_specs=pl.BlockSpec((N, OUTW), lambda i: (0, 0)),
        out_shape=jax.ShapeDtypeStruct((N, OUTW), jnp.float32),
    )(agg2p, y2, dis, b2c)


def kernel(z, edge_attr, mlp_W1, mlp_b1, mlp_W2, mlp_b2,
           conv1_W, conv1_b, conv2_W, conv2_b, edge_index):
    del edge_attr
    row = edge_index[0]
    col = edge_index[1]

    degp = _hist(col)

    x_pack = _mlp(z, mlp_W1, mlp_b1.reshape(1, MLPH), mlp_W2,
                  mlp_b2.reshape(1, N * F))
    x2d = x_pack.reshape(NP, F)

    aggp, y, dis_v = _scale_agg16(x2d, degp, row, col)
    dis = dis_v[:N].reshape(N, 1)
    y2 = _conv(aggp, y, dis,
               conv1_W, conv1_b.reshape(1, HID), conv2_W)

    agg2p = _agg32(y2, row, col)
    out = _final(agg2p, y2, dis,
                 conv2_b.reshape(1, OUTW))
    return out
